# baseline (speedup 1.0000x reference)
import functools

import jax
import jax.numpy as jnp
from jax import lax
from jax.experimental import pallas as pl
from jax.experimental.pallas import tpu as pltpu
from jax.experimental.pallas import tpu_sc as plsc

_N = 2048
_D = 768
_F = 3072
_E = 8
_BR = 256
_NB = _N // _BR
_CH = 64
_L = 16
_ZROW = _E * _N


def _router_body(x_ref, wr_ref, i0_ref, i1_ref, g0_ref, g1_ref):
    l = jnp.dot(x_ref[...], wr_ref[...], preferred_element_type=jnp.float32)
    io = lax.broadcasted_iota(jnp.int32, (_N, _E), 1)
    m1 = jnp.max(l, axis=1, keepdims=True)
    a1 = jnp.min(jnp.where(l == m1, io, _E), axis=1, keepdims=True)
    l2 = jnp.where(io == a1, -jnp.inf, l)
    m2 = jnp.max(l2, axis=1, keepdims=True)
    a2 = jnp.min(jnp.where(l2 == m2, io, _E), axis=1, keepdims=True)
    e2 = jnp.exp(m2 - m1)
    den = 1.0 + e2
    i0_ref[...] = a1[:, 0]
    i1_ref[...] = a2[:, 0]
    g0_ref[...] = (1.0 / den)[:, 0]
    g1_ref[...] = (e2 / den)[:, 0]


def _router(xf, Wr):
    return pl.pallas_call(
        _router_body,
        out_shape=(
            jax.ShapeDtypeStruct((_N,), jnp.int32),
            jax.ShapeDtypeStruct((_N,), jnp.int32),
            jax.ShapeDtypeStruct((_N,), jnp.float32),
            jax.ShapeDtypeStruct((_N,), jnp.float32),
        ),
    )(xf, Wr)


def _sc_compact_body(idx0_h, idx1_h, g0_h, g1_h, xf_h,
                     counts_h, wgt_h, pose_h, xg_h,
                     idx0_v, idx1_v, g0_v, g1_v,
                     S_v, srcp_v, wgtp_v, pose_v,
                     cnt_v, idxc_v, rows_v, sem):
    c = lax.axis_index("c")
    s = lax.axis_index("s")

    @pl.when((c == 0) & (s < _E))
    def _():
        e = s
        with jax.named_scope("cp_load"):
            pltpu.sync_copy(idx0_h, idx0_v)
            pltpu.sync_copy(idx1_h, idx1_v)
            pltpu.sync_copy(g0_h, g0_v)
            pltpu.sync_copy(g1_h, g1_v)
        iota = lax.broadcasted_iota(jnp.int32, (_L,), 0)
        zero_f = jnp.zeros((_L,), jnp.float32)
        neg1 = jnp.full((_L,), -1, jnp.int32)

        def init(j, _):
            wgtp_v[pl.ds(j * _L, _L)] = zero_f
            pose_v[pl.ds(j * _L, _L)] = neg1
            return 0

        with jax.named_scope("cp_init"):
            lax.fori_loop(0, _N // _L, init, 0)

        def pass1(j, ns):
            tok = j * _L + iota
            i0 = idx0_v[pl.ds(j * _L, _L)]
            i1 = idx1_v[pl.ds(j * _L, _L)]
            sel = (i0 == e) | (i1 == e)
            seli = sel.astype(jnp.int32)
            ranks = ns + plsc.cumsum(seli) - 1
            plsc.store_scatter(S_v, [ranks], tok, mask=sel)
            return ns + plsc.all_reduce_population_count(sel)[0]

        with jax.named_scope("cp_pass1"):
            ns = lax.fori_loop(0, _N // _L, pass1, jnp.int32(0))

        def pass2(j, p):
            tok = j * _L + iota
            i0 = idx0_v[pl.ds(j * _L, _L)]
            i1 = idx1_v[pl.ds(j * _L, _L)]
            m0 = i0 == e
            sel = m0 | (i1 == e)
            valid = sel & (tok < ns)
            vi = valid.astype(jnp.int32)
            slots = p + plsc.cumsum(vi) - 1
            srcv = plsc.load_gather(S_v, [tok])
            g = jnp.where(m0, g0_v[pl.ds(j * _L, _L)], g1_v[pl.ds(j * _L, _L)])
            plsc.store_scatter(srcp_v, [slots], srcv, mask=valid)
            plsc.store_scatter(wgtp_v, [slots], g, mask=valid)
            plsc.store_scatter(pose_v, [tok], slots, mask=valid)
            return p + plsc.all_reduce_population_count(valid)[0]

        with jax.named_scope("cp_pass2"):
            cnt = lax.fori_loop(0, _N // _L, pass2, jnp.int32(0))

        with jax.named_scope("cp_wb"):
            cnt_v[...] = jnp.full((_L,), cnt, jnp.int32)
            pltpu.sync_copy(cnt_v, counts_h.at[pl.ds(e * _L, _L)])
            pltpu.sync_copy(wgtp_v, wgt_h.at[pl.ds(e * _N, _N)])
            pltpu.sync_copy(pose_v, pose_h.at[pl.ds(e * _N, _N)])

        nblocks = (cnt + _BR - 1) // _BR
        nchunks = nblocks * (_BR // _CH)

        def gchunk(m, _):
            base = m * _CH

            def fill(j2, _2):
                lp = base + j2 * _L + iota
                v = srcp_v[pl.ds(base + j2 * _L, _L)]
                idxc_v[pl.ds(j2 * _L, _L)] = jnp.where(lp < cnt, v, 0)
                return 0

            lax.fori_loop(0, _CH // _L, fill, 0)
            pltpu.async_copy(xf_h.at[idxc_v], rows_v, sem).wait()
            pltpu.sync_copy(rows_v, xg_h.at[pl.ds(e * _N + base, _CH)])
            return 0

        with jax.named_scope("cp_gather"):
            lax.fori_loop(0, nchunks, gchunk, 0)


def _build_sc_compact(interpret=False):
    mesh = plsc.VectorSubcoreMesh(core_axis_name="c", subcore_axis_name="s")
    return functools.partial(
        pl.kernel,
        mesh=mesh,
        interpret=interpret,
        compiler_params=pltpu.CompilerParams(needs_layout_passes=False),
        out_type=(
            jax.ShapeDtypeStruct((_E * _L,), jnp.int32),
            jax.ShapeDtypeStruct((_E * _N,), jnp.float32),
            jax.ShapeDtypeStruct((_E * _N,), jnp.int32),
            jax.ShapeDtypeStruct((_E * _N, _D), jnp.float32),
        ),
        scratch_types=[
            pltpu.VMEM((_N,), jnp.int32),
            pltpu.VMEM((_N,), jnp.int32),
            pltpu.VMEM((_N,), jnp.float32),
            pltpu.VMEM((_N,), jnp.float32),
            pltpu.VMEM((_N,), jnp.int32),
            pltpu.VMEM((_N,), jnp.int32),
            pltpu.VMEM((_N,), jnp.float32),
            pltpu.VMEM((_N,), jnp.int32),
            pltpu.VMEM((_L,), jnp.int32),
            pltpu.VMEM((_CH,), jnp.int32),
            pltpu.VMEM((_CH, _D), jnp.float32),
            pltpu.SemaphoreType.DMA,
        ],
    )(_sc_compact_body)


def _ffn_body(counts_ref, xg_ref, w1_ref, b1_ref, w2_ref, b2_ref, g_ref,
              be_ref, wgt_ref, yg_ref):
    g = pl.program_id(0)
    e = jnp.minimum(g // _NB, _E - 1)
    b = g % _NB
    nb = (counts_ref[e] + _BR - 1) // _BR
    is_z = g == _E * _NB

    @pl.when(is_z)
    def _():
        yg_ref[...] = jnp.zeros((_BR, _D), jnp.float32)

    @pl.when((~is_z) & (b < nb))
    def _():
        xb = xg_ref[...]
        h = jnp.dot(xb.astype(jnp.bfloat16), w1_ref[0].astype(jnp.bfloat16),
                    preferred_element_type=jnp.float32)
        h = jnp.maximum(h + b1_ref[0, 0][None, :], 0.0)
        o = jnp.dot(h.astype(jnp.bfloat16), w2_ref[0].astype(jnp.bfloat16),
                    preferred_element_type=jnp.float32)
        o = o + b2_ref[0, 0][None, :]
        hh = xb + o
        mu = jnp.mean(hh, axis=-1, keepdims=True)
        var = jnp.mean((hh - mu) ** 2, axis=-1, keepdims=True)
        y = (hh - mu) * jax.lax.rsqrt(var + 1e-6)
        y = y * g_ref[0, 0][None, :] + be_ref[0, 0][None, :]
        yg_ref[...] = y * wgt_ref[0, 0][:, None]


def _ffn(counts, xg, W1, b1, W2, b2, gamma, beta, wgt):
    def eb(g, counts_ref):
        e = jnp.minimum(g // _NB, _E - 1)
        nb = (counts_ref[e] + _BR - 1) // _BR
        b = jnp.minimum(g % _NB, jnp.maximum(nb - 1, 0))
        return e, b

    def xg_map(g, c):
        e, b = eb(g, c)
        return (jnp.where(g == _E * _NB, _E * _NB, e * _NB + b), 0)

    def w_map(g, c):
        e, _ = eb(g, c)
        return (e, 0, 0)

    def wgt_map(g, c):
        e, b = eb(g, c)
        return (e, 0, b)

    grid_spec = pltpu.PrefetchScalarGridSpec(
        num_scalar_prefetch=1,
        grid=(_E * _NB + 1,),
        in_specs=[
            pl.BlockSpec((_BR, _D), lambda g, c: (
                jnp.minimum(xg_map(g, c)[0], _E * _NB - 1), 0)),
            pl.BlockSpec((1, _D, _F), w_map),
            pl.BlockSpec((1, 1, _F), w_map),
            pl.BlockSpec((1, _F, _D), w_map),
            pl.BlockSpec((1, 1, _D), w_map),
            pl.BlockSpec((1, 1, _D), w_map),
            pl.BlockSpec((1, 1, _D), w_map),
            pl.BlockSpec((1, 1, _BR), wgt_map),
        ],
        out_specs=pl.BlockSpec((_BR, _D), xg_map),
    )
    return pl.pallas_call(
        _ffn_body,
        grid_spec=grid_spec,
        out_shape=jax.ShapeDtypeStruct(((_E * _NB + 1) * _BR, _D), jnp.float32),
    )(counts, xg,
      W1, b1.reshape(_E, 1, _F), W2, b2.reshape(_E, 1, _D),
      gamma.reshape(_E, 1, _D), beta.reshape(_E, 1, _D),
      wgt.reshape(_E, 1, _N))


def _sc_combine_body(yg_h, pose_h, idx0_h, idx1_h, out_h,
                     pose_v, i0c_v, i1c_v, gidx0_v, gidx1_v, rows_v, rows1_v,
                     sem, sem1):
    c = lax.axis_index("c")
    s = lax.axis_index("s")
    wid = s * 2 + c
    base = wid * _CH
    iota = lax.broadcasted_iota(jnp.int32, (_L,), 0)

    with jax.named_scope("cb_pose"):
        pltpu.sync_copy(pose_h, pose_v)
        pltpu.sync_copy(idx0_h.at[pl.ds(base, _CH)], i0c_v)
        pltpu.sync_copy(idx1_h.at[pl.ds(base, _CH)], i1c_v)

    with jax.named_scope("cb_gidx"):
      for j2 in range(_CH // _L):
        tok = base + j2 * _L + iota
        i0 = i0c_v[pl.ds(j2 * _L, _L)]
        i1 = i1c_v[pl.ds(j2 * _L, _L)]
        p0 = plsc.load_gather(pose_v, [i0 * _N + tok])
        p1 = plsc.load_gather(pose_v, [i1 * _N + tok])
        gidx0_v[pl.ds(j2 * _L, _L)] = jnp.where(p0 >= 0, i0 * _N + p0, _ZROW)
        gidx1_v[pl.ds(j2 * _L, _L)] = jnp.where(p1 >= 0, i1 * _N + p1, _ZROW)

    with jax.named_scope("cb_gath"):
        d0 = pltpu.async_copy(yg_h.at[gidx0_v], rows_v, sem)
        d1 = pltpu.async_copy(yg_h.at[gidx1_v], rows1_v, sem1)
        d0.wait()
        d1.wait()

    def addj(j, _):
        off = j * _L
        for r in range(_CH):
            rows_v[r, pl.ds(off, _L)] = (rows_v[r, pl.ds(off, _L)]
                                         + rows1_v[r, pl.ds(off, _L)])
        return 0

    with jax.named_scope("cb_add"):
        lax.fori_loop(0, _D // _L, addj, 0)
    with jax.named_scope("cb_out"):
        pltpu.sync_copy(rows_v, out_h.at[pl.ds(base, _CH)])


def _build_sc_combine(interpret=False):
    mesh = plsc.VectorSubcoreMesh(core_axis_name="c", subcore_axis_name="s")
    return functools.partial(
        pl.kernel,
        mesh=mesh,
        interpret=interpret,
        compiler_params=pltpu.CompilerParams(needs_layout_passes=False),
        out_type=jax.ShapeDtypeStruct((_N, _D), jnp.float32),
        scratch_types=[
            pltpu.VMEM((_E * _N,), jnp.int32),
            pltpu.VMEM((_CH,), jnp.int32),
            pltpu.VMEM((_CH,), jnp.int32),
            pltpu.VMEM((_CH,), jnp.int32),
            pltpu.VMEM((_CH,), jnp.int32),
            pltpu.VMEM((_CH, _D), jnp.float32),
            pltpu.VMEM((_CH, _D), jnp.float32),
            pltpu.SemaphoreType.DMA,
            pltpu.SemaphoreType.DMA,
        ],
    )(_sc_combine_body)


def kernel(x, Wr, W1, b1, W2, b2, gamma, beta):
    B, S, D = x.shape
    xf = x.reshape(_N, _D)
    idx0, idx1, g0, g1 = _router(xf, Wr)
    counts16, wgt, pose, xg = _build_sc_compact()(idx0, idx1, g0, g1, xf)
    counts = counts16.reshape(_E, _L)[:, 0]
    yg = _ffn(counts, xg, W1, b1, W2, b2, gamma, beta, wgt)
    out = _build_sc_combine()(yg, pose, idx0, idx1)
    return out.reshape(B, S, D)


# speedup vs baseline: 1.2506x; 1.2506x over previous
import functools

import jax
import jax.numpy as jnp
from jax import lax
from jax.experimental import pallas as pl
from jax.experimental.pallas import tpu as pltpu
from jax.experimental.pallas import tpu_sc as plsc

_N = 2048
_D = 768
_F = 3072
_E = 8
_BR = 256
_NB = _N // _BR
_CH = 64
_L = 16
_ZROW = _E * _N


def _router_body(x_ref, wr_ref, i0_ref, i1_ref, g0_ref, g1_ref):
    l = jnp.dot(x_ref[...], wr_ref[...], preferred_element_type=jnp.float32)
    io = lax.broadcasted_iota(jnp.int32, (_N, _E), 1)
    m1 = jnp.max(l, axis=1, keepdims=True)
    a1 = jnp.min(jnp.where(l == m1, io, _E), axis=1, keepdims=True)
    l2 = jnp.where(io == a1, -jnp.inf, l)
    m2 = jnp.max(l2, axis=1, keepdims=True)
    a2 = jnp.min(jnp.where(l2 == m2, io, _E), axis=1, keepdims=True)
    e2 = jnp.exp(m2 - m1)
    den = 1.0 + e2
    i0_ref[...] = a1[:, 0]
    i1_ref[...] = a2[:, 0]
    g0_ref[...] = (1.0 / den)[:, 0]
    g1_ref[...] = (e2 / den)[:, 0]


def _router(xf, Wr):
    return pl.pallas_call(
        _router_body,
        out_shape=(
            jax.ShapeDtypeStruct((_N,), jnp.int32),
            jax.ShapeDtypeStruct((_N,), jnp.int32),
            jax.ShapeDtypeStruct((_N,), jnp.float32),
            jax.ShapeDtypeStruct((_N,), jnp.float32),
        ),
    )(xf, Wr)


def _sc_compact_body(idx0_h, idx1_h, g0_h, g1_h, xf_h,
                     counts_h, wgt_h, pose_h, destk_h, xg_h,
                     idx0_v, idx1_v, g0_v, g1_v,
                     S_v, srcp_v, wgtp_v, pose_v, destk_v,
                     cnt_v, idxc_v, rows_v, sem):
    c = lax.axis_index("c")
    s = lax.axis_index("s")

    @pl.when((c == 0) & (s < _E))
    def _():
        e = s
        with jax.named_scope("cp_load"):
            pltpu.sync_copy(idx0_h, idx0_v)
            pltpu.sync_copy(idx1_h, idx1_v)
            pltpu.sync_copy(g0_h, g0_v)
            pltpu.sync_copy(g1_h, g1_v)
        iota = lax.broadcasted_iota(jnp.int32, (_L,), 0)
        zero_f = jnp.zeros((_L,), jnp.float32)
        neg1 = jnp.full((_L,), -1, jnp.int32)

        def init(j, _):
            wgtp_v[pl.ds(j * _L, _L)] = zero_f
            pose_v[pl.ds(j * _L, _L)] = neg1
            return 0

        with jax.named_scope("cp_init"):
            lax.fori_loop(0, _N // _L, init, 0)

        def pass1(j, ns):
            tok = j * _L + iota
            i0 = idx0_v[pl.ds(j * _L, _L)]
            i1 = idx1_v[pl.ds(j * _L, _L)]
            sel = (i0 == e) | (i1 == e)
            seli = sel.astype(jnp.int32)
            ranks = ns + plsc.cumsum(seli) - 1
            plsc.store_scatter(S_v, [ranks], tok, mask=sel)
            return ns + plsc.all_reduce_population_count(sel)[0]

        with jax.named_scope("cp_pass1"):
            ns = lax.fori_loop(0, _N // _L, pass1, jnp.int32(0))

        def pass2(j, p):
            tok = j * _L + iota
            i0 = idx0_v[pl.ds(j * _L, _L)]
            i1 = idx1_v[pl.ds(j * _L, _L)]
            m0 = i0 == e
            sel = m0 | (i1 == e)
            valid = sel & (tok < ns)
            vi = valid.astype(jnp.int32)
            slots = p + plsc.cumsum(vi) - 1
            srcv = plsc.load_gather(S_v, [tok])
            g = jnp.where(m0, g0_v[pl.ds(j * _L, _L)], g1_v[pl.ds(j * _L, _L)])
            plsc.store_scatter(srcp_v, [slots], srcv, mask=valid)
            plsc.store_scatter(wgtp_v, [slots], g, mask=valid)
            plsc.store_scatter(pose_v, [tok], slots, mask=valid)
            plsc.store_scatter(destk_v, [slots],
                               jnp.where(m0, tok, tok + _N), mask=valid)
            return p + plsc.all_reduce_population_count(valid)[0]

        with jax.named_scope("cp_pass2"):
            cnt = lax.fori_loop(0, _N // _L, pass2, jnp.int32(0))

        with jax.named_scope("cp_wb"):
            cnt_v[...] = jnp.full((_L,), cnt, jnp.int32)
            pltpu.sync_copy(cnt_v, counts_h.at[pl.ds(e * _L, _L)])
            pltpu.sync_copy(wgtp_v, wgt_h.at[pl.ds(e * _N, _N)])
            pltpu.sync_copy(pose_v, pose_h.at[pl.ds(e * _N, _N)])
            pltpu.sync_copy(destk_v, destk_h.at[pl.ds(e * _N, _N)])

        nblocks = (cnt + _BR - 1) // _BR
        nchunks = nblocks * (_BR // _CH)

        def gchunk(m, _):
            base = m * _CH

            def fill(j2, _2):
                lp = base + j2 * _L + iota
                v = srcp_v[pl.ds(base + j2 * _L, _L)]
                idxc_v[pl.ds(j2 * _L, _L)] = jnp.where(lp < cnt, v, 0)
                return 0

            lax.fori_loop(0, _CH // _L, fill, 0)
            pltpu.async_copy(xf_h.at[idxc_v], rows_v, sem).wait()
            pltpu.sync_copy(rows_v, xg_h.at[pl.ds(e * _N + base, _CH)])
            return 0

        with jax.named_scope("cp_gather"):
            lax.fori_loop(0, nchunks, gchunk, 0)


def _build_sc_compact(interpret=False):
    mesh = plsc.VectorSubcoreMesh(core_axis_name="c", subcore_axis_name="s")
    return functools.partial(
        pl.kernel,
        mesh=mesh,
        interpret=interpret,
        compiler_params=pltpu.CompilerParams(needs_layout_passes=False),
        out_type=(
            jax.ShapeDtypeStruct((_E * _L,), jnp.int32),
            jax.ShapeDtypeStruct((_E * _N,), jnp.float32),
            jax.ShapeDtypeStruct((_E * _N,), jnp.int32),
            jax.ShapeDtypeStruct((_E * _N,), jnp.int32),
            jax.ShapeDtypeStruct((_E * _N, _D), jnp.float32),
        ),
        scratch_types=[
            pltpu.VMEM((_N,), jnp.int32),
            pltpu.VMEM((_N,), jnp.int32),
            pltpu.VMEM((_N,), jnp.float32),
            pltpu.VMEM((_N,), jnp.float32),
            pltpu.VMEM((_N,), jnp.int32),
            pltpu.VMEM((_N,), jnp.int32),
            pltpu.VMEM((_N,), jnp.float32),
            pltpu.VMEM((_N,), jnp.int32),
            pltpu.VMEM((_N,), jnp.int32),
            pltpu.VMEM((_L,), jnp.int32),
            pltpu.VMEM((_CH,), jnp.int32),
            pltpu.VMEM((_CH, _D), jnp.float32),
            pltpu.SemaphoreType.DMA,
        ],
    )(_sc_compact_body)


def _ffn_body(counts_ref, xg_ref, w1_ref, b1_ref, w2_ref, b2_ref, g_ref,
              be_ref, wgt_ref, yg_ref):
    g = pl.program_id(0)
    e = jnp.minimum(g // _NB, _E - 1)
    b = g % _NB
    nb = (counts_ref[e] + _BR - 1) // _BR
    is_z = g == _E * _NB

    @pl.when(is_z)
    def _():
        yg_ref[...] = jnp.zeros((_BR, _D), jnp.float32)

    @pl.when((~is_z) & (b < nb))
    def _():
        xb = xg_ref[...]
        h = jnp.dot(xb.astype(jnp.bfloat16), w1_ref[0].astype(jnp.bfloat16),
                    preferred_element_type=jnp.float32)
        h = jnp.maximum(h + b1_ref[0, 0][None, :], 0.0)
        o = jnp.dot(h.astype(jnp.bfloat16), w2_ref[0].astype(jnp.bfloat16),
                    preferred_element_type=jnp.float32)
        o = o + b2_ref[0, 0][None, :]
        hh = xb + o
        mu = jnp.mean(hh, axis=-1, keepdims=True)
        var = jnp.mean((hh - mu) ** 2, axis=-1, keepdims=True)
        y = (hh - mu) * jax.lax.rsqrt(var + 1e-6)
        y = y * g_ref[0, 0][None, :] + be_ref[0, 0][None, :]
        yg_ref[...] = y * wgt_ref[0, 0][:, None]


def _ffn(counts, xg, W1, b1, W2, b2, gamma, beta, wgt):
    def eb(g, counts_ref):
        e = jnp.minimum(g // _NB, _E - 1)
        nb = (counts_ref[e] + _BR - 1) // _BR
        b = jnp.minimum(g % _NB, jnp.maximum(nb - 1, 0))
        return e, b

    def xg_map(g, c):
        e, b = eb(g, c)
        return (jnp.where(g == _E * _NB, _E * _NB, e * _NB + b), 0)

    def w_map(g, c):
        e, _ = eb(g, c)
        return (e, 0, 0)

    def wgt_map(g, c):
        e, b = eb(g, c)
        return (e, 0, b)

    grid_spec = pltpu.PrefetchScalarGridSpec(
        num_scalar_prefetch=1,
        grid=(_E * _NB + 1,),
        in_specs=[
            pl.BlockSpec((_BR, _D), lambda g, c: (
                jnp.minimum(xg_map(g, c)[0], _E * _NB - 1), 0)),
            pl.BlockSpec((1, _D, _F), w_map),
            pl.BlockSpec((1, 1, _F), w_map),
            pl.BlockSpec((1, _F, _D), w_map),
            pl.BlockSpec((1, 1, _D), w_map),
            pl.BlockSpec((1, 1, _D), w_map),
            pl.BlockSpec((1, 1, _D), w_map),
            pl.BlockSpec((1, 1, _BR), wgt_map),
        ],
        out_specs=pl.BlockSpec((_BR, _D), xg_map),
    )
    return pl.pallas_call(
        _ffn_body,
        grid_spec=grid_spec,
        out_shape=jax.ShapeDtypeStruct(((_E * _NB + 1) * _BR, _D), jnp.float32),
    )(counts, xg,
      W1, b1.reshape(_E, 1, _F), W2, b2.reshape(_E, 1, _D),
      gamma.reshape(_E, 1, _D), beta.reshape(_E, 1, _D),
      wgt.reshape(_E, 1, _N))


_DUMMY = 2 * _N


def _sc_combine_body(yg_h, destk_h, counts_h, pose_h, idx0_h, idx1_h, out2_h,
                     pose_v, i0c_v, i1c_v, destc_v, fill_v, rows_v, zc_v,
                     cnt_v, sem, semz):
    c = lax.axis_index("c")
    s = lax.axis_index("s")
    wid = s * 2 + c
    iota = lax.broadcasted_iota(jnp.int32, (_L,), 0)
    zero_f = jnp.zeros((_L,), jnp.float32)

    def zfill(j, _):
        off = j * _L
        for r in range(_CH):
            zc_v[r, pl.ds(off, _L)] = zero_f
        return 0

    lax.fori_loop(0, _D // _L, zfill, 0)

    e = wid % _E
    q = wid // _E
    pltpu.sync_copy(counts_h.at[pl.ds(e * _L, _L)], cnt_v)
    cnt = cnt_v[...][0]
    nchunks = (cnt + _CH - 1) // _CH
    niter = jnp.maximum((nchunks - q + 3) // 4, 0)

    def chunk(i, _):
        m = q + i * 4
        base = m * _CH
        pltpu.sync_copy(yg_h.at[pl.ds(e * _N + base, _CH)], rows_v)
        pltpu.sync_copy(destk_h.at[pl.ds(e * _N + base, _CH)], destc_v)

        def fix(j2, _2):
            lp = base + j2 * _L + iota
            v = destc_v[pl.ds(j2 * _L, _L)]
            destc_v[pl.ds(j2 * _L, _L)] = jnp.where(lp < cnt, v, _DUMMY)
            return 0

        lax.fori_loop(0, _CH // _L, fix, 0)
        pltpu.async_copy(rows_v, out2_h.at[destc_v], sem).wait()
        return 0

    with jax.named_scope("cb_scat"):
        lax.fori_loop(0, niter, chunk, 0)

    base = wid * _CH
    with jax.named_scope("cb_fill"):
        pltpu.sync_copy(pose_h, pose_v)
        pltpu.sync_copy(idx0_h.at[pl.ds(base, _CH)], i0c_v)
        pltpu.sync_copy(idx1_h.at[pl.ds(base, _CH)], i1c_v)
        for k in range(2):
            ic_v = i0c_v if k == 0 else i1c_v
            for j2 in range(_CH // _L):
                tok = base + j2 * _L + iota
                ie = ic_v[pl.ds(j2 * _L, _L)]
                pk = plsc.load_gather(pose_v, [ie * _N + tok])
                fill_v[pl.ds(j2 * _L, _L)] = jnp.where(pk < 0, k * _N + tok,
                                                       _DUMMY)
            pltpu.async_copy(zc_v, out2_h.at[fill_v], semz).wait()


def _build_sc_combine(interpret=False):
    mesh = plsc.VectorSubcoreMesh(core_axis_name="c", subcore_axis_name="s")
    return functools.partial(
        pl.kernel,
        mesh=mesh,
        interpret=interpret,
        compiler_params=pltpu.CompilerParams(needs_layout_passes=False),
        out_type=jax.ShapeDtypeStruct((2 * _N + 8, _D), jnp.float32),
        scratch_types=[
            pltpu.VMEM((_E * _N,), jnp.int32),
            pltpu.VMEM((_CH,), jnp.int32),
            pltpu.VMEM((_CH,), jnp.int32),
            pltpu.VMEM((_CH,), jnp.int32),
            pltpu.VMEM((_CH,), jnp.int32),
            pltpu.VMEM((_CH, _D), jnp.float32),
            pltpu.VMEM((_CH, _D), jnp.float32),
            pltpu.VMEM((_L,), jnp.int32),
            pltpu.SemaphoreType.DMA,
            pltpu.SemaphoreType.DMA,
        ],
    )(_sc_combine_body)


def _add_body(a_ref, b_ref, o_ref):
    o_ref[...] = a_ref[...] + b_ref[...]


def _final_add(out2):
    return pl.pallas_call(
        _add_body,
        grid=(_N // _BR,),
        in_specs=[
            pl.BlockSpec((_BR, _D), lambda b: (b, 0)),
            pl.BlockSpec((_BR, _D), lambda b: (b + _N // _BR, 0)),
        ],
        out_specs=pl.BlockSpec((_BR, _D), lambda b: (b, 0)),
        out_shape=jax.ShapeDtypeStruct((_N, _D), jnp.float32),
    )(out2, out2)


def kernel(x, Wr, W1, b1, W2, b2, gamma, beta):
    B, S, D = x.shape
    xf = x.reshape(_N, _D)
    idx0, idx1, g0, g1 = _router(xf, Wr)
    counts16, wgt, pose, destk, xg = _build_sc_compact()(idx0, idx1, g0, g1, xf)
    counts = counts16.reshape(_E, _L)[:, 0]
    yg = _ffn(counts, xg, W1, b1, W2, b2, gamma, beta, wgt)
    out2 = _build_sc_combine()(yg, destk, counts16, pose, idx0, idx1)
    out = _final_add(out2)
    return out.reshape(B, S, D)


# speedup vs baseline: 1.2784x; 1.0223x over previous
import functools

import jax
import jax.numpy as jnp
from jax import lax
from jax.experimental import pallas as pl
from jax.experimental.pallas import tpu as pltpu
from jax.experimental.pallas import tpu_sc as plsc

_N = 2048
_D = 768
_F = 3072
_E = 8
_BR = 256
_NB = _N // _BR
_CH = 64
_L = 16
_ZROW = _E * _N


def _router_body(x_ref, wr_ref, i0_ref, i1_ref, g0_ref, g1_ref):
    l = jnp.dot(x_ref[...], wr_ref[...], preferred_element_type=jnp.float32)
    io = lax.broadcasted_iota(jnp.int32, (_N, _E), 1)
    m1 = jnp.max(l, axis=1, keepdims=True)
    a1 = jnp.min(jnp.where(l == m1, io, _E), axis=1, keepdims=True)
    l2 = jnp.where(io == a1, -jnp.inf, l)
    m2 = jnp.max(l2, axis=1, keepdims=True)
    a2 = jnp.min(jnp.where(l2 == m2, io, _E), axis=1, keepdims=True)
    e2 = jnp.exp(m2 - m1)
    den = 1.0 + e2
    i0_ref[...] = a1[:, 0]
    i1_ref[...] = a2[:, 0]
    g0_ref[...] = (1.0 / den)[:, 0]
    g1_ref[...] = (e2 / den)[:, 0]


def _router(xf, Wr):
    return pl.pallas_call(
        _router_body,
        out_shape=(
            jax.ShapeDtypeStruct((_N,), jnp.int32),
            jax.ShapeDtypeStruct((_N,), jnp.int32),
            jax.ShapeDtypeStruct((_N,), jnp.float32),
            jax.ShapeDtypeStruct((_N,), jnp.float32),
        ),
    )(xf, Wr)


def _sc_compact_body(idx0_h, idx1_h, g0_h, g1_h, xf_h,
                     counts_h, wgt_h, pose_h, destk_h, srcg_h, xg_h,
                     idx0_v, idx1_v, g0_v, g1_v,
                     S_v, srcp_v, wgtp_v, pose_v, destk_v,
                     cnt_v, idxc_v, rows_v, sem):
    c = lax.axis_index("c")
    s = lax.axis_index("s")
    iota = lax.broadcasted_iota(jnp.int32, (_L,), 0)

    @pl.when(s < 4)
    def _():
        e = s * 2 + c
        with jax.named_scope("cp_load"):
            pltpu.sync_copy(idx0_h, idx0_v)
            pltpu.sync_copy(idx1_h, idx1_v)
            pltpu.sync_copy(g0_h, g0_v)
            pltpu.sync_copy(g1_h, g1_v)
        zero_f = jnp.zeros((_L,), jnp.float32)
        neg1 = jnp.full((_L,), -1, jnp.int32)

        def init(j, _):
            wgtp_v[pl.ds(j * _L, _L)] = zero_f
            pose_v[pl.ds(j * _L, _L)] = neg1
            return 0

        with jax.named_scope("cp_init"):
            lax.fori_loop(0, _N // _L, init, 0)

        def pass1(j, ns):
            tok = j * _L + iota
            i0 = idx0_v[pl.ds(j * _L, _L)]
            i1 = idx1_v[pl.ds(j * _L, _L)]
            sel = (i0 == e) | (i1 == e)
            seli = sel.astype(jnp.int32)
            ranks = ns + plsc.cumsum(seli) - 1
            plsc.store_scatter(S_v, [ranks], tok, mask=sel)
            return ns + plsc.all_reduce_population_count(sel)[0]

        with jax.named_scope("cp_pass1"):
            ns = lax.fori_loop(0, _N // _L, pass1, jnp.int32(0))

        def pass2(j, p):
            tok = j * _L + iota
            i0 = idx0_v[pl.ds(j * _L, _L)]
            i1 = idx1_v[pl.ds(j * _L, _L)]
            m0 = i0 == e
            sel = m0 | (i1 == e)
            valid = sel & (tok < ns)
            vi = valid.astype(jnp.int32)
            slots = p + plsc.cumsum(vi) - 1
            srcv = plsc.load_gather(S_v, [tok])
            g = jnp.where(m0, g0_v[pl.ds(j * _L, _L)], g1_v[pl.ds(j * _L, _L)])
            plsc.store_scatter(srcp_v, [slots], srcv, mask=valid)
            plsc.store_scatter(wgtp_v, [slots], g, mask=valid)
            plsc.store_scatter(pose_v, [tok], slots, mask=valid)
            plsc.store_scatter(destk_v, [slots],
                               jnp.where(m0, tok, tok + _N), mask=valid)
            return p + plsc.all_reduce_population_count(valid)[0]

        with jax.named_scope("cp_pass2"):
            cnt = lax.fori_loop(0, _N // _L, pass2, jnp.int32(0))

        def clamp(j, _):
            slot = j * _L + iota
            v = srcp_v[pl.ds(j * _L, _L)]
            srcp_v[pl.ds(j * _L, _L)] = jnp.where(slot < cnt, v, 0)
            return 0

        lax.fori_loop(0, _N // _L, clamp, 0)

        with jax.named_scope("cp_wb"):
            cnt_v[...] = jnp.full((_L,), cnt, jnp.int32)
            pltpu.sync_copy(cnt_v, counts_h.at[pl.ds(e * _L, _L)])
            pltpu.sync_copy(wgtp_v, wgt_h.at[pl.ds(e * _N, _N)])
            pltpu.sync_copy(pose_v, pose_h.at[pl.ds(e * _N, _N)])
            pltpu.sync_copy(destk_v, destk_h.at[pl.ds(e * _N, _N)])
            pltpu.sync_copy(srcp_v, srcg_h.at[pl.ds(e * _N, _N)])

    plsc.subcore_barrier()

    e = (s % 4) * 2 + c
    q = s // 4
    pltpu.sync_copy(counts_h.at[pl.ds(e * _L, _L)], cnt_v)
    cnt = cnt_v[...][0]
    nchunks = ((cnt + _BR - 1) // _BR) * (_BR // _CH)
    niter = jnp.maximum((nchunks - q + 3) // 4, 0)

    def gchunk(i, _):
        base = (q + i * 4) * _CH
        pltpu.sync_copy(srcg_h.at[pl.ds(e * _N + base, _CH)], idxc_v)
        pltpu.async_copy(xf_h.at[idxc_v], rows_v, sem).wait()
        pltpu.sync_copy(rows_v, xg_h.at[pl.ds(e * _N + base, _CH)])
        return 0

    with jax.named_scope("cp_gather"):
        lax.fori_loop(0, niter, gchunk, 0)


def _build_sc_compact(interpret=False):
    mesh = plsc.VectorSubcoreMesh(core_axis_name="c", subcore_axis_name="s")
    return functools.partial(
        pl.kernel,
        mesh=mesh,
        interpret=interpret,
        compiler_params=pltpu.CompilerParams(needs_layout_passes=False),
        out_type=(
            jax.ShapeDtypeStruct((_E * _L,), jnp.int32),
            jax.ShapeDtypeStruct((_E * _N,), jnp.float32),
            jax.ShapeDtypeStruct((_E * _N,), jnp.int32),
            jax.ShapeDtypeStruct((_E * _N,), jnp.int32),
            jax.ShapeDtypeStruct((_E * _N,), jnp.int32),
            jax.ShapeDtypeStruct((_E * _N, _D), jnp.float32),
        ),
        scratch_types=[
            pltpu.VMEM((_N,), jnp.int32),
            pltpu.VMEM((_N,), jnp.int32),
            pltpu.VMEM((_N,), jnp.float32),
            pltpu.VMEM((_N,), jnp.float32),
            pltpu.VMEM((_N,), jnp.int32),
            pltpu.VMEM((_N,), jnp.int32),
            pltpu.VMEM((_N,), jnp.float32),
            pltpu.VMEM((_N,), jnp.int32),
            pltpu.VMEM((_N,), jnp.int32),
            pltpu.VMEM((_L,), jnp.int32),
            pltpu.VMEM((_CH,), jnp.int32),
            pltpu.VMEM((_CH, _D), jnp.float32),
            pltpu.SemaphoreType.DMA,
        ],
    )(_sc_compact_body)


def _ffn_body(counts_ref, xg_ref, w1_ref, b1_ref, w2_ref, b2_ref, g_ref,
              be_ref, wgt_ref, yg_ref):
    g = pl.program_id(0)
    e = jnp.minimum(g // _NB, _E - 1)
    b = g % _NB
    nb = (counts_ref[e] + _BR - 1) // _BR
    is_z = g == _E * _NB

    @pl.when(is_z)
    def _():
        yg_ref[...] = jnp.zeros((_BR, _D), jnp.float32)

    @pl.when((~is_z) & (b < nb))
    def _():
        xb = xg_ref[...]
        h = jnp.dot(xb.astype(jnp.bfloat16), w1_ref[0].astype(jnp.bfloat16),
                    preferred_element_type=jnp.float32)
        h = jnp.maximum(h + b1_ref[0, 0][None, :], 0.0)
        o = jnp.dot(h.astype(jnp.bfloat16), w2_ref[0].astype(jnp.bfloat16),
                    preferred_element_type=jnp.float32)
        o = o + b2_ref[0, 0][None, :]
        hh = xb + o
        mu = jnp.mean(hh, axis=-1, keepdims=True)
        var = jnp.mean((hh - mu) ** 2, axis=-1, keepdims=True)
        y = (hh - mu) * jax.lax.rsqrt(var + 1e-6)
        y = y * g_ref[0, 0][None, :] + be_ref[0, 0][None, :]
        yg_ref[...] = y * wgt_ref[0, 0][:, None]


def _ffn(counts, xg, W1, b1, W2, b2, gamma, beta, wgt):
    def eb(g, counts_ref):
        e = jnp.minimum(g // _NB, _E - 1)
        nb = (counts_ref[e] + _BR - 1) // _BR
        b = jnp.minimum(g % _NB, jnp.maximum(nb - 1, 0))
        return e, b

    def xg_map(g, c):
        e, b = eb(g, c)
        return (jnp.where(g == _E * _NB, _E * _NB, e * _NB + b), 0)

    def w_map(g, c):
        e, _ = eb(g, c)
        return (e, 0, 0)

    def wgt_map(g, c):
        e, b = eb(g, c)
        return (e, 0, b)

    grid_spec = pltpu.PrefetchScalarGridSpec(
        num_scalar_prefetch=1,
        grid=(_E * _NB + 1,),
        in_specs=[
            pl.BlockSpec((_BR, _D), lambda g, c: (
                jnp.minimum(xg_map(g, c)[0], _E * _NB - 1), 0)),
            pl.BlockSpec((1, _D, _F), w_map),
            pl.BlockSpec((1, 1, _F), w_map),
            pl.BlockSpec((1, _F, _D), w_map),
            pl.BlockSpec((1, 1, _D), w_map),
            pl.BlockSpec((1, 1, _D), w_map),
            pl.BlockSpec((1, 1, _D), w_map),
            pl.BlockSpec((1, 1, _BR), wgt_map),
        ],
        out_specs=pl.BlockSpec((_BR, _D), xg_map),
    )
    return pl.pallas_call(
        _ffn_body,
        grid_spec=grid_spec,
        out_shape=jax.ShapeDtypeStruct(((_E * _NB + 1) * _BR, _D), jnp.float32),
    )(counts, xg,
      W1, b1.reshape(_E, 1, _F), W2, b2.reshape(_E, 1, _D),
      gamma.reshape(_E, 1, _D), beta.reshape(_E, 1, _D),
      wgt.reshape(_E, 1, _N))


_DUMMY = 2 * _N


def _sc_combine_body(yg_h, destk_h, counts_h, pose_h, idx0_h, idx1_h, out2_h,
                     pose_v, i0c_v, i1c_v, destc_v, fill_v, rows_v, zc_v,
                     cnt_v, sem, semz):
    c = lax.axis_index("c")
    s = lax.axis_index("s")
    wid = s * 2 + c
    iota = lax.broadcasted_iota(jnp.int32, (_L,), 0)
    zero_f = jnp.zeros((_L,), jnp.float32)

    def zfill(j, _):
        off = j * _L
        for r in range(_CH):
            zc_v[r, pl.ds(off, _L)] = zero_f
        return 0

    lax.fori_loop(0, _D // _L, zfill, 0)

    e = wid % _E
    q = wid // _E
    pltpu.sync_copy(counts_h.at[pl.ds(e * _L, _L)], cnt_v)
    cnt = cnt_v[...][0]
    nchunks = (cnt + _CH - 1) // _CH
    niter = jnp.maximum((nchunks - q + 3) // 4, 0)

    def chunk(i, _):
        m = q + i * 4
        base = m * _CH
        pltpu.sync_copy(yg_h.at[pl.ds(e * _N + base, _CH)], rows_v)
        pltpu.sync_copy(destk_h.at[pl.ds(e * _N + base, _CH)], destc_v)

        def fix(j2, _2):
            lp = base + j2 * _L + iota
            v = destc_v[pl.ds(j2 * _L, _L)]
            destc_v[pl.ds(j2 * _L, _L)] = jnp.where(lp < cnt, v, _DUMMY)
            return 0

        lax.fori_loop(0, _CH // _L, fix, 0)
        pltpu.async_copy(rows_v, out2_h.at[destc_v], sem).wait()
        return 0

    with jax.named_scope("cb_scat"):
        lax.fori_loop(0, niter, chunk, 0)

    base = wid * _CH
    with jax.named_scope("cb_fill"):
        for e2 in range(_E):
            pltpu.sync_copy(pose_h.at[pl.ds(e2 * _N + base, _CH)],
                            pose_v.at[pl.ds(e2 * _CH, _CH)])
        pltpu.sync_copy(idx0_h.at[pl.ds(base, _CH)], i0c_v)
        pltpu.sync_copy(idx1_h.at[pl.ds(base, _CH)], i1c_v)
        for k in range(2):
            ic_v = i0c_v if k == 0 else i1c_v
            for j2 in range(_CH // _L):
                lt = j2 * _L + iota
                tok = base + lt
                ie = ic_v[pl.ds(j2 * _L, _L)]
                pk = plsc.load_gather(pose_v, [ie * _CH + lt])
                fill_v[pl.ds(j2 * _L, _L)] = jnp.where(pk < 0, k * _N + tok,
                                                       _DUMMY)
            pltpu.async_copy(zc_v, out2_h.at[fill_v], semz).wait()


def _build_sc_combine(interpret=False):
    mesh = plsc.VectorSubcoreMesh(core_axis_name="c", subcore_axis_name="s")
    return functools.partial(
        pl.kernel,
        mesh=mesh,
        interpret=interpret,
        compiler_params=pltpu.CompilerParams(needs_layout_passes=False),
        out_type=jax.ShapeDtypeStruct((2 * _N + 8, _D), jnp.float32),
        scratch_types=[
            pltpu.VMEM((_E * _CH,), jnp.int32),
            pltpu.VMEM((_CH,), jnp.int32),
            pltpu.VMEM((_CH,), jnp.int32),
            pltpu.VMEM((_CH,), jnp.int32),
            pltpu.VMEM((_CH,), jnp.int32),
            pltpu.VMEM((_CH, _D), jnp.float32),
            pltpu.VMEM((_CH, _D), jnp.float32),
            pltpu.VMEM((_L,), jnp.int32),
            pltpu.SemaphoreType.DMA,
            pltpu.SemaphoreType.DMA,
        ],
    )(_sc_combine_body)


def _add_body(a_ref, b_ref, o_ref):
    o_ref[...] = a_ref[...] + b_ref[...]


def _final_add(out2):
    return pl.pallas_call(
        _add_body,
        grid=(_N // _BR,),
        in_specs=[
            pl.BlockSpec((_BR, _D), lambda b: (b, 0)),
            pl.BlockSpec((_BR, _D), lambda b: (b + _N // _BR, 0)),
        ],
        out_specs=pl.BlockSpec((_BR, _D), lambda b: (b, 0)),
        out_shape=jax.ShapeDtypeStruct((_N, _D), jnp.float32),
    )(out2, out2)


def kernel(x, Wr, W1, b1, W2, b2, gamma, beta):
    B, S, D = x.shape
    xf = x.reshape(_N, _D)
    idx0, idx1, g0, g1 = _router(xf, Wr)
    counts16, wgt, pose, destk, _srcg, xg = _build_sc_compact()(idx0, idx1, g0, g1, xf)
    counts = counts16.reshape(_E, _L)[:, 0]
    yg = _ffn(counts, xg, W1, b1, W2, b2, gamma, beta, wgt)
    out2 = _build_sc_combine()(yg, destk, counts16, pose, idx0, idx1)
    out = _final_add(out2)
    return out.reshape(B, S, D)


# speedup vs baseline: 1.4426x; 1.1284x over previous
import functools

import jax
import jax.numpy as jnp
from jax import lax
from jax.experimental import pallas as pl
from jax.experimental.pallas import tpu as pltpu
from jax.experimental.pallas import tpu_sc as plsc

_N = 2048
_D = 768
_F = 3072
_E = 8
_BR = 256
_NB = _N // _BR
_CH = 64
_L = 16
_ZROW = _E * _N


def _router_body(x_ref, wr_ref, i0_ref, i1_ref, g0_ref, g1_ref):
    l = jnp.dot(x_ref[...], wr_ref[...], preferred_element_type=jnp.float32)
    io = lax.broadcasted_iota(jnp.int32, (_N, _E), 1)
    m1 = jnp.max(l, axis=1, keepdims=True)
    a1 = jnp.min(jnp.where(l == m1, io, _E), axis=1, keepdims=True)
    l2 = jnp.where(io == a1, -jnp.inf, l)
    m2 = jnp.max(l2, axis=1, keepdims=True)
    a2 = jnp.min(jnp.where(l2 == m2, io, _E), axis=1, keepdims=True)
    e2 = jnp.exp(m2 - m1)
    den = 1.0 + e2
    i0_ref[...] = a1[:, 0]
    i1_ref[...] = a2[:, 0]
    g0_ref[...] = (1.0 / den)[:, 0]
    g1_ref[...] = (e2 / den)[:, 0]


def _router(xf, Wr):
    return pl.pallas_call(
        _router_body,
        out_shape=(
            jax.ShapeDtypeStruct((_N,), jnp.int32),
            jax.ShapeDtypeStruct((_N,), jnp.int32),
            jax.ShapeDtypeStruct((_N,), jnp.float32),
            jax.ShapeDtypeStruct((_N,), jnp.float32),
        ),
    )(xf, Wr)


def _sc_compact_body(idx0_h, idx1_h, g0_h, g1_h, xf_h,
                     counts_h, wgt_h, pose_h, destk_h, srcg_h, xg_h,
                     idx0_v, idx1_v, g0_v, g1_v,
                     S_v, srcp_v, wgtp_v, pose_v, destk_v,
                     cnt_v, idxc_v, rows_v, sem):
    c = lax.axis_index("c")
    s = lax.axis_index("s")
    iota = lax.broadcasted_iota(jnp.int32, (_L,), 0)

    @pl.when(s < 4)
    def _():
        e = s * 2 + c
        with jax.named_scope("cp_load"):
            pltpu.sync_copy(idx0_h, idx0_v)
            pltpu.sync_copy(idx1_h, idx1_v)
            pltpu.sync_copy(g0_h, g0_v)
            pltpu.sync_copy(g1_h, g1_v)
        zero_f = jnp.zeros((_L,), jnp.float32)
        neg1 = jnp.full((_L,), -1, jnp.int32)

        def init(j, _):
            wgtp_v[pl.ds(j * _L, _L)] = zero_f
            pose_v[pl.ds(j * _L, _L)] = neg1
            return 0

        with jax.named_scope("cp_init"):
            lax.fori_loop(0, _N // _L, init, 0)

        def pass1(j, ns):
            tok = j * _L + iota
            i0 = idx0_v[pl.ds(j * _L, _L)]
            i1 = idx1_v[pl.ds(j * _L, _L)]
            sel = (i0 == e) | (i1 == e)
            seli = sel.astype(jnp.int32)
            ranks = ns + plsc.cumsum(seli) - 1
            plsc.store_scatter(S_v, [ranks], tok, mask=sel)
            return ns + plsc.all_reduce_population_count(sel)[0]

        with jax.named_scope("cp_pass1"):
            ns = lax.fori_loop(0, _N // _L, pass1, jnp.int32(0))

        def pass2(j, p):
            tok = j * _L + iota
            i0 = idx0_v[pl.ds(j * _L, _L)]
            i1 = idx1_v[pl.ds(j * _L, _L)]
            m0 = i0 == e
            sel = m0 | (i1 == e)
            valid = sel & (tok < ns)
            vi = valid.astype(jnp.int32)
            slots = p + plsc.cumsum(vi) - 1
            srcv = plsc.load_gather(S_v, [tok])
            g = jnp.where(m0, g0_v[pl.ds(j * _L, _L)], g1_v[pl.ds(j * _L, _L)])
            plsc.store_scatter(srcp_v, [slots], srcv, mask=valid)
            plsc.store_scatter(wgtp_v, [slots], g, mask=valid)
            plsc.store_scatter(pose_v, [tok], slots, mask=valid)
            plsc.store_scatter(destk_v, [slots],
                               jnp.where(m0, tok, tok + _N), mask=valid)
            return p + plsc.all_reduce_population_count(valid)[0]

        with jax.named_scope("cp_pass2"):
            cnt = lax.fori_loop(0, _N // _L, pass2, jnp.int32(0))

        def clamp(j, _):
            slot = j * _L + iota
            v = srcp_v[pl.ds(j * _L, _L)]
            srcp_v[pl.ds(j * _L, _L)] = jnp.where(slot < cnt, v, 0)
            return 0

        lax.fori_loop(0, _N // _L, clamp, 0)

        with jax.named_scope("cp_wb"):
            cnt_v[...] = jnp.full((_L,), cnt, jnp.int32)
            pltpu.sync_copy(cnt_v, counts_h.at[pl.ds(e * _L, _L)])
            pltpu.sync_copy(wgtp_v, wgt_h.at[pl.ds(e * _N, _N)])
            pltpu.sync_copy(pose_v, pose_h.at[pl.ds(e * _N, _N)])
            pltpu.sync_copy(destk_v, destk_h.at[pl.ds(e * _N, _N)])
            pltpu.sync_copy(srcp_v, srcg_h.at[pl.ds(e * _N, _N)])

    plsc.subcore_barrier()

    e = (s % 4) * 2 + c
    q = s // 4
    pltpu.sync_copy(counts_h.at[pl.ds(e * _L, _L)], cnt_v)
    cnt = cnt_v[...][0]
    nchunks = ((cnt + _BR - 1) // _BR) * (_BR // _CH)
    niter = jnp.maximum((nchunks - q + 3) // 4, 0)

    def gchunk(i, _):
        base = (q + i * 4) * _CH
        pltpu.sync_copy(srcg_h.at[pl.ds(e * _N + base, _CH)], idxc_v)
        pltpu.async_copy(xf_h.at[idxc_v], rows_v, sem).wait()
        pltpu.sync_copy(rows_v, xg_h.at[pl.ds(e * _N + base, _CH)])
        return 0

    with jax.named_scope("cp_gather"):
        lax.fori_loop(0, niter, gchunk, 0)


def _build_sc_compact(interpret=False):
    mesh = plsc.VectorSubcoreMesh(core_axis_name="c", subcore_axis_name="s")
    return functools.partial(
        pl.kernel,
        mesh=mesh,
        interpret=interpret,
        compiler_params=pltpu.CompilerParams(needs_layout_passes=False),
        out_type=(
            jax.ShapeDtypeStruct((_E * _L,), jnp.int32),
            jax.ShapeDtypeStruct((_E * _N,), jnp.float32),
            jax.ShapeDtypeStruct((_E * _N,), jnp.int32),
            jax.ShapeDtypeStruct((_E * _N,), jnp.int32),
            jax.ShapeDtypeStruct((_E * _N,), jnp.int32),
            jax.ShapeDtypeStruct((_E * _N, _D), jnp.float32),
        ),
        scratch_types=[
            pltpu.VMEM((_N,), jnp.int32),
            pltpu.VMEM((_N,), jnp.int32),
            pltpu.VMEM((_N,), jnp.float32),
            pltpu.VMEM((_N,), jnp.float32),
            pltpu.VMEM((_N,), jnp.int32),
            pltpu.VMEM((_N,), jnp.int32),
            pltpu.VMEM((_N,), jnp.float32),
            pltpu.VMEM((_N,), jnp.int32),
            pltpu.VMEM((_N,), jnp.int32),
            pltpu.VMEM((_L,), jnp.int32),
            pltpu.VMEM((_CH,), jnp.int32),
            pltpu.VMEM((_CH, _D), jnp.float32),
            pltpu.SemaphoreType.DMA,
        ],
    )(_sc_compact_body)


def _ffn_body(counts_ref, xg_ref, w1_ref, b1_ref, w2_ref, b2_ref, g_ref,
              be_ref, wgt_ref, yg_ref):
    g = pl.program_id(0)
    e = jnp.minimum(g // _NB, _E - 1)
    b = g % _NB
    nb = (counts_ref[e] + _BR - 1) // _BR
    is_z = g == _E * _NB

    @pl.when(is_z)
    def _():
        yg_ref[...] = jnp.zeros((_BR, _D), jnp.float32)

    @pl.when((~is_z) & (b < nb))
    def _():
        xb = xg_ref[...]
        h = jnp.dot(xb.astype(jnp.bfloat16), w1_ref[0].astype(jnp.bfloat16),
                    preferred_element_type=jnp.float32)
        h = jnp.maximum(h + b1_ref[0, 0][None, :], 0.0)
        o = jnp.dot(h.astype(jnp.bfloat16), w2_ref[0].astype(jnp.bfloat16),
                    preferred_element_type=jnp.float32)
        o = o + b2_ref[0, 0][None, :]
        hh = xb + o
        mu = jnp.mean(hh, axis=-1, keepdims=True)
        var = jnp.mean((hh - mu) ** 2, axis=-1, keepdims=True)
        y = (hh - mu) * jax.lax.rsqrt(var + 1e-6)
        y = y * g_ref[0, 0][None, :] + be_ref[0, 0][None, :]
        yg_ref[...] = y * wgt_ref[0, 0][:, None]


def _ffn(counts, xg, W1, b1, W2, b2, gamma, beta, wgt):
    def eb(g, counts_ref):
        e = jnp.minimum(g // _NB, _E - 1)
        nb = (counts_ref[e] + _BR - 1) // _BR
        b = jnp.minimum(g % _NB, jnp.maximum(nb - 1, 0))
        return e, b

    def xg_map(g, c):
        e, b = eb(g, c)
        return (jnp.where(g == _E * _NB, _E * _NB, e * _NB + b), 0)

    def w_map(g, c):
        e = jnp.minimum(g // _NB, _E - 1)
        b = g % _NB
        nb = (c[e] + _BR - 1) // _BR
        we = jnp.where((b < nb) & (g < _E * _NB), e, jnp.minimum(e + 1, _E - 1))
        return (we, 0, 0)

    def wgt_map(g, c):
        e, b = eb(g, c)
        return (e, 0, b)

    grid_spec = pltpu.PrefetchScalarGridSpec(
        num_scalar_prefetch=1,
        grid=(_E * _NB + 1,),
        in_specs=[
            pl.BlockSpec((_BR, _D), lambda g, c: (
                jnp.minimum(xg_map(g, c)[0], _E * _NB - 1), 0)),
            pl.BlockSpec((1, _D, _F), w_map),
            pl.BlockSpec((1, 1, _F), w_map),
            pl.BlockSpec((1, _F, _D), w_map),
            pl.BlockSpec((1, 1, _D), w_map),
            pl.BlockSpec((1, 1, _D), w_map),
            pl.BlockSpec((1, 1, _D), w_map),
            pl.BlockSpec((1, 1, _BR), wgt_map),
        ],
        out_specs=pl.BlockSpec((_BR, _D), xg_map),
    )
    return pl.pallas_call(
        _ffn_body,
        grid_spec=grid_spec,
        out_shape=jax.ShapeDtypeStruct(((_E * _NB + 1) * _BR, _D), jnp.float32),
    )(counts, xg,
      W1, b1.reshape(_E, 1, _F), W2, b2.reshape(_E, 1, _D),
      gamma.reshape(_E, 1, _D), beta.reshape(_E, 1, _D),
      wgt.reshape(_E, 1, _N))


_DUMMY = 2 * _N


def _sc_combine_body(yg_h, destk_h, counts_h, pose_h, idx0_h, idx1_h, out2_h,
                     pose_v, i0c_v, i1c_v, destc_v, fill_v, rows_v, zc_v,
                     cnt_v, sem, semz):
    c = lax.axis_index("c")
    s = lax.axis_index("s")
    wid = s * 2 + c
    iota = lax.broadcasted_iota(jnp.int32, (_L,), 0)
    zero_f = jnp.zeros((_L,), jnp.float32)

    def zfill(j, _):
        off = j * _L
        for r in range(_CH):
            zc_v[r, pl.ds(off, _L)] = zero_f
        return 0

    lax.fori_loop(0, _D // _L, zfill, 0)

    e = wid % _E
    q = wid // _E
    pltpu.sync_copy(counts_h.at[pl.ds(e * _L, _L)], cnt_v)
    cnt = cnt_v[...][0]
    nchunks = (cnt + _CH - 1) // _CH
    niter = jnp.maximum((nchunks - q + 3) // 4, 0)

    def chunk(i, _):
        m = q + i * 4
        base = m * _CH
        pltpu.sync_copy(yg_h.at[pl.ds(e * _N + base, _CH)], rows_v)
        pltpu.sync_copy(destk_h.at[pl.ds(e * _N + base, _CH)], destc_v)

        def fix(j2, _2):
            lp = base + j2 * _L + iota
            v = destc_v[pl.ds(j2 * _L, _L)]
            destc_v[pl.ds(j2 * _L, _L)] = jnp.where(lp < cnt, v, _DUMMY)
            return 0

        lax.fori_loop(0, _CH // _L, fix, 0)
        pltpu.async_copy(rows_v, out2_h.at[destc_v], sem).wait()
        return 0

    with jax.named_scope("cb_scat"):
        lax.fori_loop(0, niter, chunk, 0)

    base = wid * _CH
    with jax.named_scope("cb_fill"):
        for e2 in range(_E):
            pltpu.sync_copy(pose_h.at[pl.ds(e2 * _N + base, _CH)],
                            pose_v.at[pl.ds(e2 * _CH, _CH)])
        pltpu.sync_copy(idx0_h.at[pl.ds(base, _CH)], i0c_v)
        pltpu.sync_copy(idx1_h.at[pl.ds(base, _CH)], i1c_v)
        for k in range(2):
            ic_v = i0c_v if k == 0 else i1c_v
            for j2 in range(_CH // _L):
                lt = j2 * _L + iota
                tok = base + lt
                ie = ic_v[pl.ds(j2 * _L, _L)]
                pk = plsc.load_gather(pose_v, [ie * _CH + lt])
                fill_v[pl.ds(j2 * _L, _L)] = jnp.where(pk < 0, k * _N + tok,
                                                       _DUMMY)
            pltpu.async_copy(zc_v, out2_h.at[fill_v], semz).wait()


def _build_sc_combine(interpret=False):
    mesh = plsc.VectorSubcoreMesh(core_axis_name="c", subcore_axis_name="s")
    return functools.partial(
        pl.kernel,
        mesh=mesh,
        interpret=interpret,
        compiler_params=pltpu.CompilerParams(needs_layout_passes=False),
        out_type=jax.ShapeDtypeStruct((2 * _N + 8, _D), jnp.float32),
        scratch_types=[
            pltpu.VMEM((_E * _CH,), jnp.int32),
            pltpu.VMEM((_CH,), jnp.int32),
            pltpu.VMEM((_CH,), jnp.int32),
            pltpu.VMEM((_CH,), jnp.int32),
            pltpu.VMEM((_CH,), jnp.int32),
            pltpu.VMEM((_CH, _D), jnp.float32),
            pltpu.VMEM((_CH, _D), jnp.float32),
            pltpu.VMEM((_L,), jnp.int32),
            pltpu.SemaphoreType.DMA,
            pltpu.SemaphoreType.DMA,
        ],
    )(_sc_combine_body)


def _add_body(a_ref, b_ref, o_ref):
    o_ref[...] = a_ref[...] + b_ref[...]


def _final_add(out2):
    return pl.pallas_call(
        _add_body,
        grid=(_N // _BR,),
        in_specs=[
            pl.BlockSpec((_BR, _D), lambda b: (b, 0)),
            pl.BlockSpec((_BR, _D), lambda b: (b + _N // _BR, 0)),
        ],
        out_specs=pl.BlockSpec((_BR, _D), lambda b: (b, 0)),
        out_shape=jax.ShapeDtypeStruct((_N, _D), jnp.float32),
    )(out2, out2)


def kernel(x, Wr, W1, b1, W2, b2, gamma, beta):
    B, S, D = x.shape
    xf = x.reshape(_N, _D)
    idx0, idx1, g0, g1 = _router(xf, Wr)
    counts16, wgt, pose, destk, _srcg, xg = _build_sc_compact()(idx0, idx1, g0, g1, xf)
    counts = counts16.reshape(_E, _L)[:, 0]
    yg = _ffn(counts, xg, W1, b1, W2, b2, gamma, beta, wgt)
    out2 = _build_sc_combine()(yg, destk, counts16, pose, idx0, idx1)
    out = _final_add(out2)
    return out.reshape(B, S, D)


# speedup vs baseline: 1.5068x; 1.0445x over previous
import functools

import jax
import jax.numpy as jnp
from jax import lax
from jax.experimental import pallas as pl
from jax.experimental.pallas import tpu as pltpu
from jax.experimental.pallas import tpu_sc as plsc

_N = 2048
_D = 768
_F = 3072
_E = 8
_BR = 256
_NB = _N // _BR
_CH = 64
_CHG = 128
_L = 16
_ZROW = _E * _N


def _router_body(x_ref, wr_ref, i0_ref, i1_ref, g0_ref, g1_ref):
    l = jnp.dot(x_ref[...], wr_ref[...], preferred_element_type=jnp.float32)
    io = lax.broadcasted_iota(jnp.int32, (_N, _E), 1)
    m1 = jnp.max(l, axis=1, keepdims=True)
    a1 = jnp.min(jnp.where(l == m1, io, _E), axis=1, keepdims=True)
    l2 = jnp.where(io == a1, -jnp.inf, l)
    m2 = jnp.max(l2, axis=1, keepdims=True)
    a2 = jnp.min(jnp.where(l2 == m2, io, _E), axis=1, keepdims=True)
    e2 = jnp.exp(m2 - m1)
    den = 1.0 + e2
    i0_ref[...] = a1[:, 0]
    i1_ref[...] = a2[:, 0]
    g0_ref[...] = (1.0 / den)[:, 0]
    g1_ref[...] = (e2 / den)[:, 0]


def _router(xf, Wr):
    return pl.pallas_call(
        _router_body,
        out_shape=(
            jax.ShapeDtypeStruct((_N,), jnp.int32),
            jax.ShapeDtypeStruct((_N,), jnp.int32),
            jax.ShapeDtypeStruct((_N,), jnp.float32),
            jax.ShapeDtypeStruct((_N,), jnp.float32),
        ),
    )(xf, Wr)


def _sc_compact_body(idx0_h, idx1_h, g0_h, g1_h, xf_h,
                     counts_h, wgt_h, pose_h, destk_h, srcg_h, xg_h,
                     idx0_v, idx1_v, g0_v, g1_v,
                     S_v, srcp_v, wgtp_v, pose_v, destk_v,
                     cnt_v, idxc_v, rows_v, sem):
    c = lax.axis_index("c")
    s = lax.axis_index("s")
    iota = lax.broadcasted_iota(jnp.int32, (_L,), 0)

    @pl.when(s < 4)
    def _():
        e = s * 2 + c
        with jax.named_scope("cp_load"):
            pltpu.sync_copy(idx0_h, idx0_v)
            pltpu.sync_copy(idx1_h, idx1_v)
            pltpu.sync_copy(g0_h, g0_v)
            pltpu.sync_copy(g1_h, g1_v)
        zero_f = jnp.zeros((_L,), jnp.float32)
        neg1 = jnp.full((_L,), -1, jnp.int32)

        def init(j, _):
            wgtp_v[pl.ds(j * _L, _L)] = zero_f
            pose_v[pl.ds(j * _L, _L)] = neg1
            return 0

        with jax.named_scope("cp_init"):
            lax.fori_loop(0, _N // _L, init, 0)

        def pass1(j, ns):
            tok = j * _L + iota
            i0 = idx0_v[pl.ds(j * _L, _L)]
            i1 = idx1_v[pl.ds(j * _L, _L)]
            sel = (i0 == e) | (i1 == e)
            seli = sel.astype(jnp.int32)
            ranks = ns + plsc.cumsum(seli) - 1
            plsc.store_scatter(S_v, [ranks], tok, mask=sel)
            return ns + plsc.all_reduce_population_count(sel)[0]

        with jax.named_scope("cp_pass1"):
            ns = lax.fori_loop(0, _N // _L, pass1, jnp.int32(0))

        def pass2(j, p):
            tok = j * _L + iota
            i0 = idx0_v[pl.ds(j * _L, _L)]
            i1 = idx1_v[pl.ds(j * _L, _L)]
            m0 = i0 == e
            sel = m0 | (i1 == e)
            valid = sel & (tok < ns)
            vi = valid.astype(jnp.int32)
            slots = p + plsc.cumsum(vi) - 1
            srcv = plsc.load_gather(S_v, [tok])
            g = jnp.where(m0, g0_v[pl.ds(j * _L, _L)], g1_v[pl.ds(j * _L, _L)])
            plsc.store_scatter(srcp_v, [slots], srcv, mask=valid)
            plsc.store_scatter(wgtp_v, [slots], g, mask=valid)
            plsc.store_scatter(pose_v, [tok], slots, mask=valid)
            plsc.store_scatter(destk_v, [slots],
                               jnp.where(m0, tok, tok + _N), mask=valid)
            return p + plsc.all_reduce_population_count(valid)[0]

        with jax.named_scope("cp_pass2"):
            cnt = lax.fori_loop(0, _N // _L, pass2, jnp.int32(0))

        def clamp(j, _):
            slot = j * _L + iota
            v = srcp_v[pl.ds(j * _L, _L)]
            srcp_v[pl.ds(j * _L, _L)] = jnp.where(slot < cnt, v, 0)
            return 0

        lax.fori_loop(0, _N // _L, clamp, 0)

        with jax.named_scope("cp_wb"):
            cnt_v[...] = jnp.full((_L,), cnt, jnp.int32)
            pltpu.sync_copy(cnt_v, counts_h.at[pl.ds(e * _L, _L)])
            pltpu.sync_copy(wgtp_v, wgt_h.at[pl.ds(e * _N, _N)])
            pltpu.sync_copy(pose_v, pose_h.at[pl.ds(e * _N, _N)])
            pltpu.sync_copy(destk_v, destk_h.at[pl.ds(e * _N, _N)])
            pltpu.sync_copy(srcp_v, srcg_h.at[pl.ds(e * _N, _N)])

    plsc.subcore_barrier()

    e = (s % 4) * 2 + c
    q = s // 4
    pltpu.sync_copy(counts_h.at[pl.ds(e * _L, _L)], cnt_v)
    cnt = cnt_v[...][0]
    nchunks = (cnt + _CHG - 1) // _CHG
    niter = jnp.maximum((nchunks - q + 3) // 4, 0)

    def gchunk(i, _):
        base = (q + i * 4) * _CHG
        pltpu.sync_copy(srcg_h.at[pl.ds(e * _N + base, _CHG)], idxc_v)
        pltpu.async_copy(xf_h.at[idxc_v], rows_v, sem).wait()
        pltpu.sync_copy(rows_v, xg_h.at[pl.ds(e * _N + base, _CHG)])
        return 0

    with jax.named_scope("cp_gather"):
        lax.fori_loop(0, niter, gchunk, 0)


def _build_sc_compact(interpret=False):
    mesh = plsc.VectorSubcoreMesh(core_axis_name="c", subcore_axis_name="s")
    return functools.partial(
        pl.kernel,
        mesh=mesh,
        interpret=interpret,
        compiler_params=pltpu.CompilerParams(needs_layout_passes=False),
        out_type=(
            jax.ShapeDtypeStruct((_E * _L,), jnp.int32),
            jax.ShapeDtypeStruct((_E * _N,), jnp.float32),
            jax.ShapeDtypeStruct((_E * _N,), jnp.int32),
            jax.ShapeDtypeStruct((_E * _N,), jnp.int32),
            jax.ShapeDtypeStruct((_E * _N,), jnp.int32),
            jax.ShapeDtypeStruct((_E * _N, _D), jnp.float32),
        ),
        scratch_types=[
            pltpu.VMEM((_N,), jnp.int32),
            pltpu.VMEM((_N,), jnp.int32),
            pltpu.VMEM((_N,), jnp.float32),
            pltpu.VMEM((_N,), jnp.float32),
            pltpu.VMEM((_N,), jnp.int32),
            pltpu.VMEM((_N,), jnp.int32),
            pltpu.VMEM((_N,), jnp.float32),
            pltpu.VMEM((_N,), jnp.int32),
            pltpu.VMEM((_N,), jnp.int32),
            pltpu.VMEM((_L,), jnp.int32),
            pltpu.VMEM((_CHG,), jnp.int32),
            pltpu.VMEM((_CHG, _D), jnp.float32),
            pltpu.SemaphoreType.DMA,
        ],
    )(_sc_compact_body)


def _ffn_body(counts_ref, xg_ref, w1_ref, b1_ref, w2_ref, b2_ref, g_ref,
              be_ref, wgt_ref, yg_ref):
    g = pl.program_id(0)
    e = jnp.minimum(g // _NB, _E - 1)
    b = g % _NB
    nb = (counts_ref[e] + _BR - 1) // _BR
    is_z = g == _E * _NB

    @pl.when(is_z)
    def _():
        yg_ref[...] = jnp.zeros((_BR, _D), jnp.float32)

    @pl.when((~is_z) & (b < nb))
    def _():
        xb = xg_ref[...]
        h = jnp.dot(xb.astype(jnp.bfloat16), w1_ref[0].astype(jnp.bfloat16),
                    preferred_element_type=jnp.float32)
        h = jnp.maximum(h + b1_ref[0, 0][None, :], 0.0)
        o = jnp.dot(h.astype(jnp.bfloat16), w2_ref[0].astype(jnp.bfloat16),
                    preferred_element_type=jnp.float32)
        o = o + b2_ref[0, 0][None, :]
        hh = xb + o
        mu = jnp.mean(hh, axis=-1, keepdims=True)
        var = jnp.mean((hh - mu) ** 2, axis=-1, keepdims=True)
        y = (hh - mu) * jax.lax.rsqrt(var + 1e-6)
        y = y * g_ref[0, 0][None, :] + be_ref[0, 0][None, :]
        yg_ref[...] = y * wgt_ref[0, 0][:, None]


def _ffn(counts, xg, W1, b1, W2, b2, gamma, beta, wgt):
    def eb(g, counts_ref):
        e = jnp.minimum(g // _NB, _E - 1)
        nb = (counts_ref[e] + _BR - 1) // _BR
        b = jnp.minimum(g % _NB, jnp.maximum(nb - 1, 0))
        return e, b

    def xg_map(g, c):
        e, b = eb(g, c)
        return (jnp.where(g == _E * _NB, _E * _NB, e * _NB + b), 0)

    def w_map(g, c):
        e = jnp.minimum(g // _NB, _E - 1)
        b = g % _NB
        nb = (c[e] + _BR - 1) // _BR
        we = jnp.where((b < nb) & (g < _E * _NB), e, jnp.minimum(e + 1, _E - 1))
        return (we, 0, 0)

    def wgt_map(g, c):
        e, b = eb(g, c)
        return (e, 0, b)

    grid_spec = pltpu.PrefetchScalarGridSpec(
        num_scalar_prefetch=1,
        grid=(_E * _NB + 1,),
        in_specs=[
            pl.BlockSpec((_BR, _D), lambda g, c: (
                jnp.minimum(xg_map(g, c)[0], _E * _NB - 1), 0)),
            pl.BlockSpec((1, _D, _F), w_map),
            pl.BlockSpec((1, 1, _F), w_map),
            pl.BlockSpec((1, _F, _D), w_map),
            pl.BlockSpec((1, 1, _D), w_map),
            pl.BlockSpec((1, 1, _D), w_map),
            pl.BlockSpec((1, 1, _D), w_map),
            pl.BlockSpec((1, 1, _BR), wgt_map),
        ],
        out_specs=pl.BlockSpec((_BR, _D), xg_map),
    )
    return pl.pallas_call(
        _ffn_body,
        grid_spec=grid_spec,
        out_shape=jax.ShapeDtypeStruct(((_E * _NB + 1) * _BR, _D), jnp.float32),
    )(counts, xg,
      W1, b1.reshape(_E, 1, _F), W2, b2.reshape(_E, 1, _D),
      gamma.reshape(_E, 1, _D), beta.reshape(_E, 1, _D),
      wgt.reshape(_E, 1, _N))


_DUMMY = 2 * _N


def _sc_combine_body(yg_h, destk_h, counts_h, pose_h, idx0_h, idx1_h, out2_h,
                     pose_v, i0c_v, i1c_v, destc_v, fill_v, rows_v,
                     cnt_v, sem, semz):
    c = lax.axis_index("c")
    s = lax.axis_index("s")
    wid = s * 2 + c
    iota = lax.broadcasted_iota(jnp.int32, (_L,), 0)
    zero_f = jnp.zeros((_L,), jnp.float32)

    e = wid % _E
    q = wid // _E
    pltpu.sync_copy(counts_h.at[pl.ds(e * _L, _L)], cnt_v)
    cnt = cnt_v[...][0]
    nchunks = (cnt + _CHG - 1) // _CHG
    niter = jnp.maximum((nchunks - q + 3) // 4, 0)

    def chunk(i, _):
        m = q + i * 4
        base = m * _CHG
        pltpu.sync_copy(yg_h.at[pl.ds(e * _N + base, _CHG)], rows_v)
        pltpu.sync_copy(destk_h.at[pl.ds(e * _N + base, _CHG)], destc_v)

        def fix(j2, _2):
            lp = base + j2 * _L + iota
            v = destc_v[pl.ds(j2 * _L, _L)]
            destc_v[pl.ds(j2 * _L, _L)] = jnp.where(lp < cnt, v, _DUMMY)
            return 0

        lax.fori_loop(0, _CHG // _L, fix, 0)
        pltpu.async_copy(rows_v, out2_h.at[destc_v], sem).wait()
        return 0

    with jax.named_scope("cb_scat"):
        lax.fori_loop(0, niter, chunk, 0)

    def zfill(j, _):
        off = j * _L
        for r in range(_CH):
            rows_v[r, pl.ds(off, _L)] = zero_f
        return 0

    lax.fori_loop(0, _D // _L, zfill, 0)

    base = wid * _CH
    with jax.named_scope("cb_fill"):
        for e2 in range(_E):
            pltpu.sync_copy(pose_h.at[pl.ds(e2 * _N + base, _CH)],
                            pose_v.at[pl.ds(e2 * _CH, _CH)])
        pltpu.sync_copy(idx0_h.at[pl.ds(base, _CH)], i0c_v)
        pltpu.sync_copy(idx1_h.at[pl.ds(base, _CH)], i1c_v)
        for k in range(2):
            ic_v = i0c_v if k == 0 else i1c_v
            for j2 in range(_CH // _L):
                lt = j2 * _L + iota
                tok = base + lt
                ie = ic_v[pl.ds(j2 * _L, _L)]
                pk = plsc.load_gather(pose_v, [ie * _CH + lt])
                fill_v[pl.ds(j2 * _L, _L)] = jnp.where(pk < 0, k * _N + tok,
                                                       _DUMMY)
            pltpu.async_copy(rows_v.at[pl.ds(0, _CH)], out2_h.at[fill_v],
                             semz).wait()


def _build_sc_combine(interpret=False):
    mesh = plsc.VectorSubcoreMesh(core_axis_name="c", subcore_axis_name="s")
    return functools.partial(
        pl.kernel,
        mesh=mesh,
        interpret=interpret,
        compiler_params=pltpu.CompilerParams(needs_layout_passes=False),
        out_type=jax.ShapeDtypeStruct((2 * _N + 8, _D), jnp.float32),
        scratch_types=[
            pltpu.VMEM((_E * _CH,), jnp.int32),
            pltpu.VMEM((_CH,), jnp.int32),
            pltpu.VMEM((_CH,), jnp.int32),
            pltpu.VMEM((_CHG,), jnp.int32),
            pltpu.VMEM((_CH,), jnp.int32),
            pltpu.VMEM((_CHG, _D), jnp.float32),
            pltpu.VMEM((_L,), jnp.int32),
            pltpu.SemaphoreType.DMA,
            pltpu.SemaphoreType.DMA,
        ],
    )(_sc_combine_body)


def _add_body(a_ref, b_ref, o_ref):
    o_ref[...] = a_ref[...] + b_ref[...]


def _final_add(out2):
    return pl.pallas_call(
        _add_body,
        grid=(_N // _BR,),
        in_specs=[
            pl.BlockSpec((_BR, _D), lambda b: (b, 0)),
            pl.BlockSpec((_BR, _D), lambda b: (b + _N // _BR, 0)),
        ],
        out_specs=pl.BlockSpec((_BR, _D), lambda b: (b, 0)),
        out_shape=jax.ShapeDtypeStruct((_N, _D), jnp.float32),
    )(out2, out2)


def kernel(x, Wr, W1, b1, W2, b2, gamma, beta):
    B, S, D = x.shape
    xf = x.reshape(_N, _D)
    idx0, idx1, g0, g1 = _router(xf, Wr)
    counts16, wgt, pose, destk, _srcg, xg = _build_sc_compact()(idx0, idx1, g0, g1, xf)
    counts = counts16.reshape(_E, _L)[:, 0]
    yg = _ffn(counts, xg, W1, b1, W2, b2, gamma, beta, wgt)
    out2 = _build_sc_combine()(yg, destk, counts16, pose, idx0, idx1)
    out = _final_add(out2)
    return out.reshape(B, S, D)


# speedup vs baseline: 1.5127x; 1.0039x over previous
import functools

import jax
import jax.numpy as jnp
from jax import lax
from jax.experimental import pallas as pl
from jax.experimental.pallas import tpu as pltpu
from jax.experimental.pallas import tpu_sc as plsc

_N = 2048
_D = 768
_F = 3072
_E = 8
_BR = 256
_NB = _N // _BR
_CH = 64
_CHG = 128
_L = 16
_ZROW = _E * _N


def _router_body(x_ref, wr_ref, i0_ref, i1_ref, g0_ref, g1_ref):
    l = jnp.dot(x_ref[...], wr_ref[...], preferred_element_type=jnp.float32)
    io = lax.broadcasted_iota(jnp.int32, (_N, _E), 1)
    m1 = jnp.max(l, axis=1, keepdims=True)
    a1 = jnp.min(jnp.where(l == m1, io, _E), axis=1, keepdims=True)
    l2 = jnp.where(io == a1, -jnp.inf, l)
    m2 = jnp.max(l2, axis=1, keepdims=True)
    a2 = jnp.min(jnp.where(l2 == m2, io, _E), axis=1, keepdims=True)
    e2 = jnp.exp(m2 - m1)
    den = 1.0 + e2
    i0_ref[...] = a1[:, 0]
    i1_ref[...] = a2[:, 0]
    g0_ref[...] = (1.0 / den)[:, 0]
    g1_ref[...] = (e2 / den)[:, 0]


def _router(xf, Wr):
    return pl.pallas_call(
        _router_body,
        out_shape=(
            jax.ShapeDtypeStruct((_N,), jnp.int32),
            jax.ShapeDtypeStruct((_N,), jnp.int32),
            jax.ShapeDtypeStruct((_N,), jnp.float32),
            jax.ShapeDtypeStruct((_N,), jnp.float32),
        ),
    )(xf, Wr)


def _sc_compact_body(idx0_h, idx1_h, g0_h, g1_h, xf_h,
                     counts_h, wgt_h, pose_h, destk_h, srcg_h, xg_h,
                     idx0_v, idx1_v, g0_v, g1_v,
                     S_v, srcp_v, wgtp_v, pose_v, destk_v,
                     cnt_v, idxc_v, rows_v, sem):
    c = lax.axis_index("c")
    s = lax.axis_index("s")
    iota = lax.broadcasted_iota(jnp.int32, (_L,), 0)

    @pl.when(s < 4)
    def _():
        e = s * 2 + c
        with jax.named_scope("cp_load"):
            pltpu.sync_copy(idx0_h, idx0_v)
            pltpu.sync_copy(idx1_h, idx1_v)
            pltpu.sync_copy(g0_h, g0_v)
            pltpu.sync_copy(g1_h, g1_v)
        zero_f = jnp.zeros((_L,), jnp.float32)
        neg1 = jnp.full((_L,), -1, jnp.int32)

        def init(j, _):
            wgtp_v[pl.ds(j * _L, _L)] = zero_f
            pose_v[pl.ds(j * _L, _L)] = neg1
            return 0

        with jax.named_scope("cp_init"):
            lax.fori_loop(0, _N // _L, init, 0)

        def pass1(j, ns):
            tok = j * _L + iota
            i0 = idx0_v[pl.ds(j * _L, _L)]
            i1 = idx1_v[pl.ds(j * _L, _L)]
            sel = (i0 == e) | (i1 == e)
            seli = sel.astype(jnp.int32)
            ranks = ns + plsc.cumsum(seli) - 1
            plsc.store_scatter(S_v, [ranks], tok, mask=sel)
            return ns + plsc.all_reduce_population_count(sel)[0]

        with jax.named_scope("cp_pass1"):
            ns = lax.fori_loop(0, _N // _L, pass1, jnp.int32(0))

        def pass2(j, p):
            tok = j * _L + iota
            i0 = idx0_v[pl.ds(j * _L, _L)]
            i1 = idx1_v[pl.ds(j * _L, _L)]
            m0 = i0 == e
            sel = m0 | (i1 == e)
            valid = sel & (tok < ns)
            vi = valid.astype(jnp.int32)
            slots = p + plsc.cumsum(vi) - 1
            srcv = plsc.load_gather(S_v, [tok])
            g = jnp.where(m0, g0_v[pl.ds(j * _L, _L)], g1_v[pl.ds(j * _L, _L)])
            plsc.store_scatter(srcp_v, [slots], srcv, mask=valid)
            plsc.store_scatter(wgtp_v, [slots], g, mask=valid)
            plsc.store_scatter(pose_v, [tok], slots, mask=valid)
            plsc.store_scatter(destk_v, [slots],
                               jnp.where(m0, tok, tok + _N), mask=valid)
            return p + plsc.all_reduce_population_count(valid)[0]

        with jax.named_scope("cp_pass2"):
            cnt = lax.fori_loop(0, _N // _L, pass2, jnp.int32(0))

        def clamp(j, _):
            slot = j * _L + iota
            v = srcp_v[pl.ds(j * _L, _L)]
            srcp_v[pl.ds(j * _L, _L)] = jnp.where(slot < cnt, v, 0)
            return 0

        lax.fori_loop(0, _N // _L, clamp, 0)

        with jax.named_scope("cp_wb"):
            cnt_v[...] = jnp.full((_L,), cnt, jnp.int32)
            pltpu.sync_copy(cnt_v, counts_h.at[pl.ds(e * _L, _L)])
            pltpu.sync_copy(wgtp_v, wgt_h.at[pl.ds(e * _N, _N)])
            pltpu.sync_copy(pose_v, pose_h.at[pl.ds(e * _N, _N)])
            pltpu.sync_copy(destk_v, destk_h.at[pl.ds(e * _N, _N)])
            pltpu.sync_copy(srcp_v, srcg_h.at[pl.ds(e * _N, _N)])

    plsc.subcore_barrier()

    e = (s % 4) * 2 + c
    q = s // 4
    pltpu.sync_copy(counts_h.at[pl.ds(e * _L, _L)], cnt_v)
    cnt = cnt_v[...][0]
    nchunks = (cnt + _CHG - 1) // _CHG
    niter = jnp.maximum((nchunks - q + 3) // 4, 0)

    def gchunk(i, _):
        base = (q + i * 4) * _CHG
        pltpu.sync_copy(srcg_h.at[pl.ds(e * _N + base, _CHG)], idxc_v)
        pltpu.async_copy(xf_h.at[idxc_v], rows_v, sem).wait()
        pltpu.sync_copy(rows_v, xg_h.at[pl.ds(e * _N + base, _CHG)])
        return 0

    with jax.named_scope("cp_gather"):
        lax.fori_loop(0, niter, gchunk, 0)


def _build_sc_compact(interpret=False):
    mesh = plsc.VectorSubcoreMesh(core_axis_name="c", subcore_axis_name="s")
    return functools.partial(
        pl.kernel,
        mesh=mesh,
        interpret=interpret,
        compiler_params=pltpu.CompilerParams(needs_layout_passes=False),
        out_type=(
            jax.ShapeDtypeStruct((_E * _L,), jnp.int32),
            jax.ShapeDtypeStruct((_E * _N,), jnp.float32),
            jax.ShapeDtypeStruct((_E * _N,), jnp.int32),
            jax.ShapeDtypeStruct((_E * _N,), jnp.int32),
            jax.ShapeDtypeStruct((_E * _N,), jnp.int32),
            jax.ShapeDtypeStruct((_E * _N, _D), jnp.float32),
        ),
        scratch_types=[
            pltpu.VMEM((_N,), jnp.int32),
            pltpu.VMEM((_N,), jnp.int32),
            pltpu.VMEM((_N,), jnp.float32),
            pltpu.VMEM((_N,), jnp.float32),
            pltpu.VMEM((_N,), jnp.int32),
            pltpu.VMEM((_N,), jnp.int32),
            pltpu.VMEM((_N,), jnp.float32),
            pltpu.VMEM((_N,), jnp.int32),
            pltpu.VMEM((_N,), jnp.int32),
            pltpu.VMEM((_L,), jnp.int32),
            pltpu.VMEM((_CHG,), jnp.int32),
            pltpu.VMEM((_CHG, _D), jnp.float32),
            pltpu.SemaphoreType.DMA,
        ],
    )(_sc_compact_body)


def _ffn_body(counts_ref, xg_ref, w1_ref, b1_ref, w2_ref, b2_ref, g_ref,
              be_ref, wgt_ref, yg_ref):
    g = pl.program_id(0)
    e = jnp.minimum(g // _NB, _E - 1)
    b = g % _NB
    nb = (counts_ref[e] + _BR - 1) // _BR
    is_z = g == _E * _NB

    @pl.when(is_z)
    def _():
        yg_ref[...] = jnp.zeros((_BR, _D), jnp.float32)

    @pl.when((~is_z) & (b < nb))
    def _():
        xb = xg_ref[...]
        h = jnp.dot(xb.astype(jnp.bfloat16), w1_ref[0].astype(jnp.bfloat16),
                    preferred_element_type=jnp.float32)
        h = jnp.maximum(h + b1_ref[0, 0][None, :], 0.0)
        o = jnp.dot(h.astype(jnp.bfloat16), w2_ref[0].astype(jnp.bfloat16),
                    preferred_element_type=jnp.float32)
        o = o + b2_ref[0, 0][None, :]
        hh = xb + o
        mu = jnp.mean(hh, axis=-1, keepdims=True)
        var = jnp.mean((hh - mu) ** 2, axis=-1, keepdims=True)
        y = (hh - mu) * jax.lax.rsqrt(var + 1e-6)
        y = y * g_ref[0, 0][None, :] + be_ref[0, 0][None, :]
        yg_ref[...] = y * wgt_ref[0, 0][:, None]


def _ffn(counts, xg, W1, b1, W2, b2, gamma, beta, wgt):
    def eb(g, counts_ref):
        e = jnp.minimum(g // _NB, _E - 1)
        nb = (counts_ref[e] + _BR - 1) // _BR
        b = jnp.minimum(g % _NB, jnp.maximum(nb - 1, 0))
        return e, b

    def xg_map(g, c):
        e, b = eb(g, c)
        return (jnp.where(g == _E * _NB, _E * _NB, e * _NB + b), 0)

    def w_map(g, c):
        e = jnp.minimum(g // _NB, _E - 1)
        b = g % _NB
        nb = (c[e] + _BR - 1) // _BR
        we = jnp.where((b < nb) & (g < _E * _NB), e, jnp.minimum(e + 1, _E - 1))
        return (we, 0, 0)

    def wgt_map(g, c):
        e, b = eb(g, c)
        return (e, 0, b)

    grid_spec = pltpu.PrefetchScalarGridSpec(
        num_scalar_prefetch=1,
        grid=(_E * _NB + 1,),
        in_specs=[
            pl.BlockSpec((_BR, _D), lambda g, c: (
                jnp.minimum(xg_map(g, c)[0], _E * _NB - 1), 0)),
            pl.BlockSpec((1, _D, _F), w_map),
            pl.BlockSpec((1, 1, _F), w_map),
            pl.BlockSpec((1, _F, _D), w_map),
            pl.BlockSpec((1, 1, _D), w_map),
            pl.BlockSpec((1, 1, _D), w_map),
            pl.BlockSpec((1, 1, _D), w_map),
            pl.BlockSpec((1, 1, _BR), wgt_map),
        ],
        out_specs=pl.BlockSpec((_BR, _D), xg_map),
    )
    return pl.pallas_call(
        _ffn_body,
        grid_spec=grid_spec,
        out_shape=jax.ShapeDtypeStruct(((_E * _NB + 1) * _BR, _D), jnp.float32),
    )(counts, xg,
      W1, b1.reshape(_E, 1, _F), W2, b2.reshape(_E, 1, _D),
      gamma.reshape(_E, 1, _D), beta.reshape(_E, 1, _D),
      wgt.reshape(_E, 1, _N))


_DUMMY = 2 * _N


def _sc_combine_body(yg_h, destk_h, counts_h, pose_h, idx0_h, idx1_h, out2_h,
                     pose_v, i0c_v, i1c_v, destc_v, fill_v, rows_v,
                     cnt_v, sem, semz):
    c = lax.axis_index("c")
    s = lax.axis_index("s")
    wid = s * 2 + c
    iota = lax.broadcasted_iota(jnp.int32, (_L,), 0)
    zero_f = jnp.zeros((_L,), jnp.float32)

    e = wid % _E
    q = wid // _E
    pltpu.sync_copy(counts_h.at[pl.ds(e * _L, _L)], cnt_v)
    cnt = cnt_v[...][0]
    nchunks = (cnt + _CHG - 1) // _CHG
    niter = jnp.maximum((nchunks - q + 3) // 4, 0)

    def chunk(i, _):
        m = q + i * 4
        base = m * _CHG
        d0 = pltpu.async_copy(yg_h.at[pl.ds(e * _N + base, _CHG)], rows_v, sem)
        d1 = pltpu.async_copy(destk_h.at[pl.ds(e * _N + base, _CHG)], destc_v,
                              semz)
        d0.wait()
        d1.wait()

        def fix(j2, _2):
            lp = base + j2 * _L + iota
            v = destc_v[pl.ds(j2 * _L, _L)]
            destc_v[pl.ds(j2 * _L, _L)] = jnp.where(lp < cnt, v, _DUMMY)
            return 0

        lax.fori_loop(0, _CHG // _L, fix, 0)
        pltpu.async_copy(rows_v, out2_h.at[destc_v], sem).wait()
        return 0

    with jax.named_scope("cb_scat"):
        lax.fori_loop(0, niter, chunk, 0)

    def zfill(j, _):
        off = j * _L
        for r in range(_CH):
            rows_v[r, pl.ds(off, _L)] = zero_f
        return 0

    lax.fori_loop(0, _D // _L, zfill, 0)

    base = wid * _CH
    with jax.named_scope("cb_fill"):
        ds = [pltpu.async_copy(pose_h.at[pl.ds(e2 * _N + base, _CH)],
                               pose_v.at[pl.ds(e2 * _CH, _CH)], semz)
              for e2 in range(_E)]
        ds.append(pltpu.async_copy(idx0_h.at[pl.ds(base, _CH)], i0c_v, semz))
        ds.append(pltpu.async_copy(idx1_h.at[pl.ds(base, _CH)], i1c_v, semz))
        for d in ds:
            d.wait()
        for k in range(2):
            ic_v = i0c_v if k == 0 else i1c_v
            for j2 in range(_CH // _L):
                lt = j2 * _L + iota
                tok = base + lt
                ie = ic_v[pl.ds(j2 * _L, _L)]
                pk = plsc.load_gather(pose_v, [ie * _CH + lt])
                fill_v[pl.ds(j2 * _L, _L)] = jnp.where(pk < 0, k * _N + tok,
                                                       _DUMMY)
            pltpu.async_copy(rows_v.at[pl.ds(0, _CH)], out2_h.at[fill_v],
                             semz).wait()


def _build_sc_combine(interpret=False):
    mesh = plsc.VectorSubcoreMesh(core_axis_name="c", subcore_axis_name="s")
    return functools.partial(
        pl.kernel,
        mesh=mesh,
        interpret=interpret,
        compiler_params=pltpu.CompilerParams(needs_layout_passes=False),
        out_type=jax.ShapeDtypeStruct((2 * _N + 8, _D), jnp.float32),
        scratch_types=[
            pltpu.VMEM((_E * _CH,), jnp.int32),
            pltpu.VMEM((_CH,), jnp.int32),
            pltpu.VMEM((_CH,), jnp.int32),
            pltpu.VMEM((_CHG,), jnp.int32),
            pltpu.VMEM((_CH,), jnp.int32),
            pltpu.VMEM((_CHG, _D), jnp.float32),
            pltpu.VMEM((_L,), jnp.int32),
            pltpu.SemaphoreType.DMA,
            pltpu.SemaphoreType.DMA,
        ],
    )(_sc_combine_body)


def _add_body(a_ref, b_ref, o_ref):
    o_ref[...] = a_ref[...] + b_ref[...]


def _final_add(out2):
    return pl.pallas_call(
        _add_body,
        grid=(_N // _BR,),
        in_specs=[
            pl.BlockSpec((_BR, _D), lambda b: (b, 0)),
            pl.BlockSpec((_BR, _D), lambda b: (b + _N // _BR, 0)),
        ],
        out_specs=pl.BlockSpec((_BR, _D), lambda b: (b, 0)),
        out_shape=jax.ShapeDtypeStruct((_N, _D), jnp.float32),
    )(out2, out2)


def kernel(x, Wr, W1, b1, W2, b2, gamma, beta):
    B, S, D = x.shape
    xf = x.reshape(_N, _D)
    idx0, idx1, g0, g1 = _router(xf, Wr)
    counts16, wgt, pose, destk, _srcg, xg = _build_sc_compact()(idx0, idx1, g0, g1, xf)
    counts = counts16.reshape(_E, _L)[:, 0]
    yg = _ffn(counts, xg, W1, b1, W2, b2, gamma, beta, wgt)
    out2 = _build_sc_combine()(yg, destk, counts16, pose, idx0, idx1)
    out = _final_add(out2)
    return out.reshape(B, S, D)


# speedup vs baseline: 1.5263x; 1.0090x over previous
import functools

import jax
import jax.numpy as jnp
from jax import lax
from jax.experimental import pallas as pl
from jax.experimental.pallas import tpu as pltpu
from jax.experimental.pallas import tpu_sc as plsc

_N = 2048
_D = 768
_F = 3072
_E = 8
_BR = 256
_NB = _N // _BR
_CH = 64
_CHG = 128
_L = 16


def _router_body(x_ref, wr_ref, i0_ref, i1_ref, g0_ref, g1_ref):
    l = jnp.dot(x_ref[...], wr_ref[...], preferred_element_type=jnp.float32)
    io = lax.broadcasted_iota(jnp.int32, (_N, _E), 1)
    m1 = jnp.max(l, axis=1, keepdims=True)
    a1 = jnp.min(jnp.where(l == m1, io, _E), axis=1, keepdims=True)
    l2 = jnp.where(io == a1, -jnp.inf, l)
    m2 = jnp.max(l2, axis=1, keepdims=True)
    a2 = jnp.min(jnp.where(l2 == m2, io, _E), axis=1, keepdims=True)
    e2 = jnp.exp(m2 - m1)
    den = 1.0 + e2
    i0_ref[...] = a1[:, 0]
    i1_ref[...] = a2[:, 0]
    g0_ref[...] = (1.0 / den)[:, 0]
    g1_ref[...] = (e2 / den)[:, 0]


def _router(xf, Wr):
    return pl.pallas_call(
        _router_body,
        out_shape=(
            jax.ShapeDtypeStruct((_N,), jnp.int32),
            jax.ShapeDtypeStruct((_N,), jnp.int32),
            jax.ShapeDtypeStruct((_N,), jnp.float32),
            jax.ShapeDtypeStruct((_N,), jnp.float32),
        ),
    )(xf, Wr)


def _sc_compact_body(idx0_h, idx1_h, g0_h, g1_h, xf_h,
                     counts_h, wgt_h, pose_h, destk_h, srcg_h, xg_h,
                     idx0_v, idx1_v, g0_v, g1_v,
                     S_v, srcp_v, wgtp_v, pose_v, destk_v,
                     cnt_v, idxc_v, rows_v, sem):
    c = lax.axis_index("c")
    s = lax.axis_index("s")
    iota = lax.broadcasted_iota(jnp.int32, (_L,), 0)

    @pl.when(s < 4)
    def _():
        e = s * 2 + c
        with jax.named_scope("cp_load"):
            pltpu.sync_copy(idx0_h, idx0_v)
            pltpu.sync_copy(idx1_h, idx1_v)
            pltpu.sync_copy(g0_h, g0_v)
            pltpu.sync_copy(g1_h, g1_v)
        zero_f = jnp.zeros((_L,), jnp.float32)
        neg1 = jnp.full((_L,), -1, jnp.int32)

        def init(j, _):
            wgtp_v[pl.ds(j * _L, _L)] = zero_f
            pose_v[pl.ds(j * _L, _L)] = neg1
            return 0

        with jax.named_scope("cp_init"):
            lax.fori_loop(0, _N // _L, init, 0)

        def pass1(j, ns):
            tok = j * _L + iota
            i0 = idx0_v[pl.ds(j * _L, _L)]
            i1 = idx1_v[pl.ds(j * _L, _L)]
            sel = (i0 == e) | (i1 == e)
            seli = sel.astype(jnp.int32)
            ranks = ns + plsc.cumsum(seli) - 1
            plsc.store_scatter(S_v, [ranks], tok, mask=sel)
            return ns + plsc.all_reduce_population_count(sel)[0]

        with jax.named_scope("cp_pass1"):
            ns = lax.fori_loop(0, _N // _L, pass1, jnp.int32(0))

        def pass2(j, p):
            tok = j * _L + iota
            i0 = idx0_v[pl.ds(j * _L, _L)]
            i1 = idx1_v[pl.ds(j * _L, _L)]
            m0 = i0 == e
            sel = m0 | (i1 == e)
            valid = sel & (tok < ns)
            vi = valid.astype(jnp.int32)
            slots = p + plsc.cumsum(vi) - 1
            srcv = plsc.load_gather(S_v, [tok])
            g = jnp.where(m0, g0_v[pl.ds(j * _L, _L)], g1_v[pl.ds(j * _L, _L)])
            plsc.store_scatter(srcp_v, [slots], srcv, mask=valid)
            plsc.store_scatter(wgtp_v, [slots], g, mask=valid)
            plsc.store_scatter(pose_v, [tok], slots, mask=valid)
            plsc.store_scatter(destk_v, [slots],
                               jnp.where(m0, tok, tok + _N), mask=valid)
            return p + plsc.all_reduce_population_count(valid)[0]

        with jax.named_scope("cp_pass2"):
            cnt = lax.fori_loop(0, _N // _L, pass2, jnp.int32(0))

        def clamp(j, _):
            slot = j * _L + iota
            v = srcp_v[pl.ds(j * _L, _L)]
            srcp_v[pl.ds(j * _L, _L)] = jnp.where(slot < cnt, v, 0)
            return 0

        lax.fori_loop(0, _N // _L, clamp, 0)

        with jax.named_scope("cp_wb"):
            cnt_v[...] = jnp.full((_L,), cnt, jnp.int32)
            pltpu.sync_copy(cnt_v, counts_h.at[pl.ds(e * _L, _L)])
            pltpu.sync_copy(wgtp_v, wgt_h.at[pl.ds(e * _N, _N)])
            pltpu.sync_copy(pose_v, pose_h.at[pl.ds(e * _N, _N)])
            pltpu.sync_copy(destk_v, destk_h.at[pl.ds(e * _N, _N)])
            pltpu.sync_copy(srcp_v, srcg_h.at[pl.ds(e * _N, _N)])

    plsc.subcore_barrier()

    e = (s % 4) * 2 + c
    q = s // 4
    pltpu.sync_copy(counts_h.at[pl.ds(e * _L, _L)], cnt_v)
    cnt = cnt_v[...][0]
    nchunks = (cnt + _CHG - 1) // _CHG
    niter = jnp.maximum((nchunks - q + 3) // 4, 0)

    def gchunk(i, _):
        base = (q + i * 4) * _CHG
        pltpu.sync_copy(srcg_h.at[pl.ds(e * _N + base, _CHG)], idxc_v)
        pltpu.async_copy(xf_h.at[idxc_v], rows_v, sem).wait()
        pltpu.sync_copy(rows_v, xg_h.at[pl.ds(e * _N + base, _CHG)])
        return 0

    with jax.named_scope("cp_gather"):
        lax.fori_loop(0, niter, gchunk, 0)


def _build_sc_compact(interpret=False):
    mesh = plsc.VectorSubcoreMesh(core_axis_name="c", subcore_axis_name="s")
    return functools.partial(
        pl.kernel,
        mesh=mesh,
        interpret=interpret,
        compiler_params=pltpu.CompilerParams(needs_layout_passes=False),
        out_type=(
            jax.ShapeDtypeStruct((_E * _L,), jnp.int32),
            jax.ShapeDtypeStruct((_E * _N,), jnp.float32),
            jax.ShapeDtypeStruct((_E * _N,), jnp.int32),
            jax.ShapeDtypeStruct((_E * _N,), jnp.int32),
            jax.ShapeDtypeStruct((_E * _N,), jnp.int32),
            jax.ShapeDtypeStruct((_E * _N, _D), jnp.float32),
        ),
        scratch_types=[
            pltpu.VMEM((_N,), jnp.int32),
            pltpu.VMEM((_N,), jnp.int32),
            pltpu.VMEM((_N,), jnp.float32),
            pltpu.VMEM((_N,), jnp.float32),
            pltpu.VMEM((_N,), jnp.int32),
            pltpu.VMEM((_N,), jnp.int32),
            pltpu.VMEM((_N,), jnp.float32),
            pltpu.VMEM((_N,), jnp.int32),
            pltpu.VMEM((_N,), jnp.int32),
            pltpu.VMEM((_L,), jnp.int32),
            pltpu.VMEM((_CHG,), jnp.int32),
            pltpu.VMEM((_CHG, _D), jnp.float32),
            pltpu.SemaphoreType.DMA,
        ],
    )(_sc_compact_body)


def _ffn_body(counts_ref, xg_ref, w1_ref, b1_ref, w2_ref, b2_ref, g_ref,
              be_ref, wgt_ref, yg_ref):
    g = pl.program_id(0)
    e = g // _NB
    b = g % _NB
    nb = (counts_ref[e] + _BR - 1) // _BR

    @pl.when(b < nb)
    def _():
        xb = xg_ref[...]
        h = jnp.dot(xb.astype(jnp.bfloat16), w1_ref[0].astype(jnp.bfloat16),
                    preferred_element_type=jnp.float32)
        h = jnp.maximum(h + b1_ref[0, 0][None, :], 0.0)
        o = jnp.dot(h.astype(jnp.bfloat16), w2_ref[0].astype(jnp.bfloat16),
                    preferred_element_type=jnp.float32)
        o = o + b2_ref[0, 0][None, :]
        hh = xb + o
        mu = jnp.mean(hh, axis=-1, keepdims=True)
        var = jnp.mean((hh - mu) ** 2, axis=-1, keepdims=True)
        y = (hh - mu) * jax.lax.rsqrt(var + 1e-6)
        y = y * g_ref[0, 0][None, :] + be_ref[0, 0][None, :]
        yg_ref[...] = y * wgt_ref[0, 0][:, None]


def _ffn(counts, xg, W1, b1, W2, b2, gamma, beta, wgt):
    def eb(g, counts_ref):
        e = g // _NB
        nb = (counts_ref[e] + _BR - 1) // _BR
        b = jnp.minimum(g % _NB, jnp.maximum(nb - 1, 0))
        return e, b

    def xg_map(g, c):
        e, b = eb(g, c)
        return (e * _NB + b, 0)

    def w_map(g, c):
        e = g // _NB
        b = g % _NB
        nb = (c[e] + _BR - 1) // _BR
        we = jnp.where(b < nb, e, jnp.minimum(e + 1, _E - 1))
        return (we, 0, 0)

    def wgt_map(g, c):
        e, b = eb(g, c)
        return (e, 0, b)

    grid_spec = pltpu.PrefetchScalarGridSpec(
        num_scalar_prefetch=1,
        grid=(_E * _NB,),
        in_specs=[
            pl.BlockSpec((_BR, _D), xg_map),
            pl.BlockSpec((1, _D, _F), w_map),
            pl.BlockSpec((1, 1, _F), w_map),
            pl.BlockSpec((1, _F, _D), w_map),
            pl.BlockSpec((1, 1, _D), w_map),
            pl.BlockSpec((1, 1, _D), w_map),
            pl.BlockSpec((1, 1, _D), w_map),
            pl.BlockSpec((1, 1, _BR), wgt_map),
        ],
        out_specs=pl.BlockSpec((_BR, _D), xg_map),
    )
    return pl.pallas_call(
        _ffn_body,
        grid_spec=grid_spec,
        out_shape=jax.ShapeDtypeStruct((_E * _NB * _BR, _D), jnp.float32),
    )(counts, xg,
      W1, b1.reshape(_E, 1, _F), W2, b2.reshape(_E, 1, _D),
      gamma.reshape(_E, 1, _D), beta.reshape(_E, 1, _D),
      wgt.reshape(_E, 1, _N))


_DUMMY = 2 * _N


def _sc_combine_body(yg_h, destk_h, counts_h, pose_h, idx0_h, idx1_h, out2_h,
                     pose_v, i0c_v, i1c_v, destc_v, fill_v, rows_v,
                     cnt_v, sem, semz):
    c = lax.axis_index("c")
    s = lax.axis_index("s")
    wid = s * 2 + c
    iota = lax.broadcasted_iota(jnp.int32, (_L,), 0)
    zero_f = jnp.zeros((_L,), jnp.float32)

    e = wid % _E
    q = wid // _E
    pltpu.sync_copy(counts_h.at[pl.ds(e * _L, _L)], cnt_v)
    cnt = cnt_v[...][0]
    nchunks = (cnt + _CHG - 1) // _CHG
    niter = jnp.maximum((nchunks - q + 3) // 4, 0)

    def chunk(i, _):
        m = q + i * 4
        base = m * _CHG
        d0 = pltpu.async_copy(yg_h.at[pl.ds(e * _N + base, _CHG)], rows_v, sem)
        d1 = pltpu.async_copy(destk_h.at[pl.ds(e * _N + base, _CHG)], destc_v,
                              semz)
        d0.wait()
        d1.wait()

        def fix(j2, _2):
            lp = base + j2 * _L + iota
            v = destc_v[pl.ds(j2 * _L, _L)]
            destc_v[pl.ds(j2 * _L, _L)] = jnp.where(lp < cnt, v, _DUMMY)
            return 0

        lax.fori_loop(0, _CHG // _L, fix, 0)
        pltpu.async_copy(rows_v, out2_h.at[destc_v], sem).wait()
        return 0

    with jax.named_scope("cb_scat"):
        lax.fori_loop(0, niter, chunk, 0)

    def zfill(j, _):
        off = j * _L
        for r in range(_CH):
            rows_v[r, pl.ds(off, _L)] = zero_f
        return 0

    lax.fori_loop(0, _D // _L, zfill, 0)

    base = wid * _CH
    with jax.named_scope("cb_fill"):
        ds = [pltpu.async_copy(pose_h.at[pl.ds(e2 * _N + base, _CH)],
                               pose_v.at[pl.ds(e2 * _CH, _CH)], semz)
              for e2 in range(_E)]
        ds.append(pltpu.async_copy(idx0_h.at[pl.ds(base, _CH)], i0c_v, semz))
        ds.append(pltpu.async_copy(idx1_h.at[pl.ds(base, _CH)], i1c_v, semz))
        for d in ds:
            d.wait()
        for k in range(2):
            ic_v = i0c_v if k == 0 else i1c_v
            for j2 in range(_CH // _L):
                lt = j2 * _L + iota
                tok = base + lt
                ie = ic_v[pl.ds(j2 * _L, _L)]
                pk = plsc.load_gather(pose_v, [ie * _CH + lt])
                fill_v[pl.ds(j2 * _L, _L)] = jnp.where(pk < 0, k * _N + tok,
                                                       _DUMMY)
            pltpu.async_copy(rows_v.at[pl.ds(0, _CH)], out2_h.at[fill_v],
                             semz).wait()


def _build_sc_combine(interpret=False):
    mesh = plsc.VectorSubcoreMesh(core_axis_name="c", subcore_axis_name="s")
    return functools.partial(
        pl.kernel,
        mesh=mesh,
        interpret=interpret,
        compiler_params=pltpu.CompilerParams(needs_layout_passes=False),
        out_type=jax.ShapeDtypeStruct((2 * _N + 8, _D), jnp.float32),
        scratch_types=[
            pltpu.VMEM((_E * _CH,), jnp.int32),
            pltpu.VMEM((_CH,), jnp.int32),
            pltpu.VMEM((_CH,), jnp.int32),
            pltpu.VMEM((_CHG,), jnp.int32),
            pltpu.VMEM((_CH,), jnp.int32),
            pltpu.VMEM((_CHG, _D), jnp.float32),
            pltpu.VMEM((_L,), jnp.int32),
            pltpu.SemaphoreType.DMA,
            pltpu.SemaphoreType.DMA,
        ],
    )(_sc_combine_body)


def _add_body(a_ref, b_ref, o_ref):
    o_ref[...] = a_ref[...] + b_ref[...]


def _final_add(out2):
    return pl.pallas_call(
        _add_body,
        grid=(_N // _BR,),
        in_specs=[
            pl.BlockSpec((_BR, _D), lambda b: (b, 0)),
            pl.BlockSpec((_BR, _D), lambda b: (b + _N // _BR, 0)),
        ],
        out_specs=pl.BlockSpec((_BR, _D), lambda b: (b, 0)),
        out_shape=jax.ShapeDtypeStruct((_N, _D), jnp.float32),
    )(out2, out2)


def kernel(x, Wr, W1, b1, W2, b2, gamma, beta):
    B, S, D = x.shape
    xf = x.reshape(_N, _D)
    idx0, idx1, g0, g1 = _router(xf, Wr)
    counts16, wgt, pose, destk, _srcg, xg = _build_sc_compact()(idx0, idx1, g0, g1, xf)
    counts = counts16.reshape(_E, _L)[:, 0]
    yg = _ffn(counts, xg, W1, b1, W2, b2, gamma, beta, wgt)
    out2 = _build_sc_combine()(yg, destk, counts16, pose, idx0, idx1)
    out = _final_add(out2)
    return out.reshape(B, S, D)
